# baseline (speedup 1.0000x reference)
import functools

import jax
import jax.numpy as jnp
from jax import lax
from jax.experimental import pallas as pl
from jax.experimental.pallas import tpu as pltpu
from jax.experimental.pallas import tpu_sc as plsc

N = 10000
E = 160000
D = 256
EMB_N = 32
EMB_D = 64
HID = 512
TIN = D + EMB_D

ROWS = 400
GRID = N // ROWS

NSUB = 16
CH = 128
NCH_SUB = 80
E_PAD = NSUB * NCH_SUB * CH
NP = 10240
ROWS_PER_SUB = NP // NSUB

IDXG = 40
PW1 = 80
PW2 = 128

_HIGH = jax.lax.Precision.HIGHEST


def _dot(a, b):
    return jax.lax.dot_general(a, b, (((1,), (0,)), ((), ())),
                               precision=_HIGH,
                               preferred_element_type=jnp.float32)


def _gelu(h):
    return 0.5 * h * (1.0 + lax.erf(h * 0.7071067811865476))


def _stage_a1(x, iso_idx_r, iso_embed):
    def body(x_ref, idx_ref, emb_ref, q0_ref, q1_ref, q2_ref, q3_ref):
        xb = x_ref[...]
        idx = jnp.reshape(idx_ref[0, 0, :], (ROWS, 1))
        onehot = (lax.broadcasted_iota(jnp.int32, (ROWS, EMB_N), 1)
                  == idx).astype(jnp.float32)
        e = _dot(onehot, emb_ref[...])
        q0_ref[...] = xb[:, 0:80]
        q1_ref[...] = xb[:, 80:160]
        q2_ref[...] = xb[:, 160:240]
        q3_ref[...] = jnp.concatenate([xb[:, 240:256], e], axis=1)

    return pl.pallas_call(
        body,
        grid=(GRID,),
        in_specs=[
            pl.BlockSpec((ROWS, D), lambda i: (i, 0)),
            pl.BlockSpec((1, 1, ROWS), lambda i: (i, 0, 0)),
            pl.BlockSpec((EMB_N, EMB_D), lambda i: (0, 0)),
        ],
        out_specs=[pl.BlockSpec((ROWS, PW1), lambda i: (i, 0))] * 4,
        out_shape=[jax.ShapeDtypeStruct((N, PW1), jnp.float32)] * 4,
    )(x, iso_idx_r, iso_embed)


def _stage_a2(q0, q1, q2, q3, W1r):
    def body(q0_ref, q1_ref, q2_ref, q3_ref, w_ref, o_ref):
        o_ref[...] = (_dot(q0_ref[...], w_ref[0:80, :])
                      + _dot(q1_ref[...], w_ref[80:160, :])
                      + _dot(q2_ref[...], w_ref[160:240, :])
                      + _dot(q3_ref[...], w_ref[240:320, :]))

    return pl.pallas_call(
        body,
        grid=(GRID,),
        in_specs=[pl.BlockSpec((ROWS, PW1), lambda i: (i, 0))] * 4
        + [pl.BlockSpec((TIN, HID), lambda i: (0, 0))],
        out_specs=pl.BlockSpec((ROWS, HID), lambda i: (i, 0)),
        out_shape=jax.ShapeDtypeStruct((N, HID), jnp.float32),
    )(q0, q1, q2, q3, W1r)


def _stage_b1(b0, b1, b2, b3, deg16, xw1r, W1l, b1l_r):
    def body(b0_ref, b1_ref, b2_ref, b3_ref, deg_ref, xw_ref, w_ref,
             bias_ref, p0_ref, p1_ref, p2_ref, p3_ref):
        inv = 1.0 / jnp.maximum(deg_ref[:, 0:1], 1.0)
        h = (_dot(b0_ref[...] * inv, w_ref[0:80, :])
             + _dot(b1_ref[...] * inv, w_ref[80:160, :])
             + _dot(b2_ref[...] * inv, w_ref[160:240, :])
             + _dot(b3_ref[...] * inv, w_ref[240:320, :])
             + xw_ref[...] + bias_ref[...])
        h = _gelu(h)
        p0_ref[...] = h[:, 0:128]
        p1_ref[...] = h[:, 128:256]
        p2_ref[...] = h[:, 256:384]
        p3_ref[...] = h[:, 384:512]

    return pl.pallas_call(
        body,
        grid=(GRID,),
        in_specs=[pl.BlockSpec((ROWS, PW1), lambda i: (i, 0))] * 4
        + [
            pl.BlockSpec((ROWS, 16), lambda i: (i, 0)),
            pl.BlockSpec((ROWS, HID), lambda i: (i, 0)),
            pl.BlockSpec((TIN, HID), lambda i: (0, 0)),
            pl.BlockSpec((1, HID), lambda i: (0, 0)),
        ],
        out_specs=[pl.BlockSpec((ROWS, PW2), lambda i: (i, 0))] * 4,
        out_shape=[jax.ShapeDtypeStruct((N, PW2), jnp.float32)] * 4,
    )(b0, b1, b2, b3, deg16, xw1r, W1l, b1l_r)


def _stage_b2(p0, p1, p2, p3, W2r):
    def body(p0_ref, p1_ref, p2_ref, p3_ref, w_ref, o_ref):
        o_ref[...] = (_dot(p0_ref[...], w_ref[0:128, :])
                      + _dot(p1_ref[...], w_ref[128:256, :])
                      + _dot(p2_ref[...], w_ref[256:384, :])
                      + _dot(p3_ref[...], w_ref[384:512, :]))

    return pl.pallas_call(
        body,
        grid=(GRID,),
        in_specs=[pl.BlockSpec((ROWS, PW2), lambda i: (i, 0))] * 4
        + [pl.BlockSpec((HID, HID), lambda i: (0, 0))],
        out_specs=pl.BlockSpec((ROWS, HID), lambda i: (i, 0)),
        out_shape=jax.ShapeDtypeStruct((N, HID), jnp.float32),
    )(p0, p1, p2, p3, W2r)


def _stage_c(a0, a1, a2, a3, deg16, h1w2r, W2l, b2l_r, Wh1, bh1_r,
             ln_g_r, ln_b_r, Wh2, bh2_r):
    def body(a0_ref, a1_ref, a2_ref, a3_ref, deg_ref, xw_ref, w2l_ref,
             b2l_ref, wh1_ref, bh1_ref, g_ref, b_ref, wh2_ref, bh2_ref,
             o_ref):
        inv = 1.0 / jnp.maximum(deg_ref[:, 0:1], 1.0)
        h = (_dot(a0_ref[...] * inv, w2l_ref[0:128, :])
             + _dot(a1_ref[...] * inv, w2l_ref[128:256, :])
             + _dot(a2_ref[...] * inv, w2l_ref[256:384, :])
             + _dot(a3_ref[...] * inv, w2l_ref[384:512, :])
             + xw_ref[...] + b2l_ref[...])
        h = _gelu(h)
        t = _dot(h, wh1_ref[...]) + bh1_ref[...]
        mu = jnp.mean(t, axis=-1, keepdims=True)
        var = jnp.mean((t - mu) ** 2, axis=-1, keepdims=True)
        t = (t - mu) * lax.rsqrt(var + 1e-5) * g_ref[...] + b_ref[...]
        t = _gelu(t)
        o_ref[...] = _dot(t, wh2_ref[...]) + bh2_ref[...]

    return pl.pallas_call(
        body,
        grid=(GRID,),
        in_specs=[pl.BlockSpec((ROWS, PW2), lambda i: (i, 0))] * 4
        + [
            pl.BlockSpec((ROWS, 16), lambda i: (i, 0)),
            pl.BlockSpec((ROWS, HID), lambda i: (i, 0)),
            pl.BlockSpec((HID, HID), lambda i: (0, 0)),
            pl.BlockSpec((1, HID), lambda i: (0, 0)),
            pl.BlockSpec((HID, 64), lambda i: (0, 0)),
            pl.BlockSpec((1, 64), lambda i: (0, 0)),
            pl.BlockSpec((1, 64), lambda i: (0, 0)),
            pl.BlockSpec((1, 64), lambda i: (0, 0)),
            pl.BlockSpec((64, 10), lambda i: (0, 0)),
            pl.BlockSpec((1, 10), lambda i: (0, 0)),
        ],
        out_specs=pl.BlockSpec((ROWS, 10), lambda i: (i, 0)),
        out_shape=jax.ShapeDtypeStruct((N, 10), jnp.float32),
    )(a0, a1, a2, a3, deg16, h1w2r, W2l, b2l_r, Wh1, bh1_r,
      ln_g_r, ln_b_r, Wh2, bh2_r)


def _mesh():
    return plsc.VectorSubcoreMesh(core_axis_name="c", subcore_axis_name="s")


_SC_PARAMS = pltpu.CompilerParams(use_tc_tiling_on_sc=False)


def _edge_loop(h_r, acc_s, sidx_v, didx_v, msg_a, msg_b, sem_a, sem_b,
               nch, dacc_s=None, ones_v=None):
    def fire(j, buf, sem):
        pltpu.async_copy(h_r.at[sidx_v.at[j]], buf, sem)

    def wait(j, buf, sem):
        pltpu.make_async_copy(h_r.at[sidx_v.at[j]], buf, sem).wait()

    def scat(j, buf):
        pltpu.sync_copy(buf, acc_s.at[didx_v.at[j]], add=True)
        if dacc_s is not None:
            pltpu.sync_copy(ones_v, dacc_s.at[didx_v.at[j]], add=True)

    fire(0, msg_a, sem_a)
    fire(1, msg_b, sem_b)

    @pl.loop(0, (nch - 2) // 2)
    def _(t):
        j = 2 * t
        wait(j, msg_a, sem_a)
        scat(j, msg_a)
        fire(j + 2, msg_a, sem_a)
        wait(j + 1, msg_b, sem_b)
        scat(j + 1, msg_b)
        fire(j + 3, msg_b, sem_b)

    wait(nch - 2, msg_a, sem_a)
    scat(nch - 2, msg_a)
    wait(nch - 1, msg_b, sem_b)
    scat(nch - 1, msg_b)


def _sc_agg1(src2, dst2, q0, q1, q2, q3, zacc, z16, ones16):
    @functools.partial(
        pl.kernel,
        out_type=[jax.ShapeDtypeStruct((NP, PW1), jnp.float32)] * 4
        + [jax.ShapeDtypeStruct((NP, 16), jnp.float32)],
        mesh=_mesh(),
        compiler_params=_SC_PARAMS,
        scratch_types=[
            pltpu.VMEM_SHARED((NP, PW1), jnp.float32),
            pltpu.VMEM_SHARED((NP, 16), jnp.float32),
            pltpu.VMEM((NCH_SUB, CH), jnp.int32),
            pltpu.VMEM((NCH_SUB, CH), jnp.int32),
            pltpu.VMEM((CH, PW1), jnp.float32),
            pltpu.VMEM((CH, PW1), jnp.float32),
            pltpu.VMEM((CH, 16), jnp.float32),
            pltpu.SemaphoreType.DMA,
            pltpu.SemaphoreType.DMA,
        ],
    )
    def k(src_r, dst_r, q0_r, q1_r, q2_r, q3_r, zacc_r, z16_r, ones_r,
          o0_r, o1_r, o2_r, o3_r, deg_r,
          acc_s, dacc_s, sidx_v, didx_v, msg_a, msg_b, ones_v,
          sem_a, sem_b):
        c = lax.axis_index("c")
        s = lax.axis_index("s")
        rbase = s * ROWS_PER_SUB
        cbase = s * NCH_SUB
        pltpu.sync_copy(src_r.at[pl.ds(cbase, NCH_SUB)], sidx_v)
        pltpu.sync_copy(dst_r.at[pl.ds(cbase, NCH_SUB)], didx_v)
        pltpu.sync_copy(ones_r, ones_v)

        def one_pass(h_r, o_r, with_deg):
            pltpu.sync_copy(zacc_r, acc_s.at[pl.ds(rbase, ROWS_PER_SUB)])
            if with_deg:
                pltpu.sync_copy(z16_r, dacc_s.at[pl.ds(rbase, ROWS_PER_SUB)])
            plsc.subcore_barrier()
            if with_deg:
                _edge_loop(h_r, acc_s, sidx_v, didx_v, msg_a, msg_b,
                           sem_a, sem_b, NCH_SUB, dacc_s, ones_v)
            else:
                _edge_loop(h_r, acc_s, sidx_v, didx_v, msg_a, msg_b,
                           sem_a, sem_b, NCH_SUB)

            plsc.subcore_barrier()
            pltpu.sync_copy(acc_s.at[pl.ds(rbase, ROWS_PER_SUB)],
                            o_r.at[pl.ds(rbase, ROWS_PER_SUB)])
            if with_deg:
                pltpu.sync_copy(dacc_s.at[pl.ds(rbase, ROWS_PER_SUB)],
                                deg_r.at[pl.ds(rbase, ROWS_PER_SUB)])
            plsc.subcore_barrier()

        @pl.when(c == 0)
        def _():
            one_pass(q0_r, o0_r, True)
            one_pass(q1_r, o1_r, False)

        @pl.when(c == 1)
        def _():
            one_pass(q2_r, o2_r, False)
            one_pass(q3_r, o3_r, False)

    return k(src2, dst2, q0, q1, q2, q3, zacc, z16, ones16)


def _sc_agg2(src2, dst2, p0, p1, p2, p3, z128):
    @functools.partial(
        pl.kernel,
        out_type=[jax.ShapeDtypeStruct((NP, PW2), jnp.float32)] * 4,
        mesh=_mesh(),
        compiler_params=_SC_PARAMS,
        scratch_types=[
            pltpu.VMEM_SHARED((NP, PW2), jnp.float32),
            pltpu.VMEM((IDXG, CH), jnp.int32),
            pltpu.VMEM((IDXG, CH), jnp.int32),
            pltpu.VMEM((CH, PW2), jnp.float32),
            pltpu.VMEM((CH, PW2), jnp.float32),
            pltpu.SemaphoreType.DMA,
            pltpu.SemaphoreType.DMA,
        ],
    )
    def k(src_r, dst_r, p0_r, p1_r, p2_r, p3_r, z128_r,
          o0_r, o1_r, o2_r, o3_r,
          acc_s, sidx_v, didx_v, msg_a, msg_b, sem_a, sem_b):
        c = lax.axis_index("c")
        s = lax.axis_index("s")
        rbase = s * ROWS_PER_SUB
        cbase = s * NCH_SUB

        def one_pass(h_r, o_r):
            pltpu.sync_copy(z128_r, acc_s.at[pl.ds(rbase, ROWS_PER_SUB)])
            plsc.subcore_barrier()
            for g in range(NCH_SUB // IDXG):
                pltpu.sync_copy(src_r.at[pl.ds(cbase + g * IDXG, IDXG)],
                                sidx_v)
                pltpu.sync_copy(dst_r.at[pl.ds(cbase + g * IDXG, IDXG)],
                                didx_v)
                _edge_loop(h_r, acc_s, sidx_v, didx_v, msg_a, msg_b,
                           sem_a, sem_b, IDXG)
            plsc.subcore_barrier()
            pltpu.sync_copy(acc_s.at[pl.ds(rbase, ROWS_PER_SUB)],
                            o_r.at[pl.ds(rbase, ROWS_PER_SUB)])
            plsc.subcore_barrier()

        @pl.when(c == 0)
        def _():
            one_pass(p0_r, o0_r)
            one_pass(p1_r, o1_r)

        @pl.when(c == 1)
        def _():
            one_pass(p2_r, o2_r)
            one_pass(p3_r, o3_r)

    return k(src2, dst2, p0, p1, p2, p3, z128)


def kernel(x, edge_index, iso_idx, iso_embed, W1l, b1l, W1r, W2l, b2l, W2r,
           Wh1, bh1, ln_g, ln_b, Wh2, bh2):
    src = edge_index[0].astype(jnp.int32)
    dst = edge_index[1].astype(jnp.int32)
    src2 = jnp.concatenate(
        [src, jnp.zeros((E_PAD - E,), jnp.int32)]).reshape(E_PAD // CH, CH)
    pad_dst = N + jnp.arange(E_PAD - E, dtype=jnp.int32) % (NP - N)
    dst2 = jnp.concatenate([dst, pad_dst]).reshape(E_PAD // CH, CH)
    iso_idx_r = iso_idx.astype(jnp.int32).reshape(GRID, 1, ROWS)

    z80 = jnp.zeros((ROWS_PER_SUB, PW1), jnp.float32)
    z128 = jnp.zeros((ROWS_PER_SUB, PW2), jnp.float32)
    z16 = jnp.zeros((ROWS_PER_SUB, 16), jnp.float32)
    ones16 = jnp.ones((CH, 16), jnp.float32)

    q0, q1, q2, q3 = _stage_a1(x, iso_idx_r, iso_embed)
    b0, b1, b2, b3, deg16 = _sc_agg1(src2, dst2, q0, q1, q2, q3,
                                     z80, z16, ones16)
    xw1r = _stage_a2(q0, q1, q2, q3, W1r)
    p0, p1, p2, p3 = _stage_b1(b0, b1, b2, b3, deg16, xw1r, W1l,
                               b1l.reshape(1, HID))
    a0, a1, a2, a3 = _sc_agg2(src2, dst2, p0, p1, p2, p3, z128)
    h1w2r = _stage_b2(p0, p1, p2, p3, W2r)
    out = _stage_c(a0, a1, a2, a3, deg16, h1w2r, W2l, b2l.reshape(1, HID),
                   Wh1, bh1.reshape(1, 64), ln_g.reshape(1, 64),
                   ln_b.reshape(1, 64), Wh2, bh2.reshape(1, 10))
    return out


# speedup vs baseline: 1.4070x; 1.4070x over previous
import functools

import jax
import jax.numpy as jnp
from jax import lax
from jax.experimental import pallas as pl
from jax.experimental.pallas import tpu as pltpu
from jax.experimental.pallas import tpu_sc as plsc

N = 10000
E = 160000
D = 256
EMB_N = 32
EMB_D = 64
HID = 512
TIN = D + EMB_D

ROWS = 400
GRID = N // ROWS

NSUB = 16
CH = 80
NCH_SUB = 126
E_PAD = NSUB * NCH_SUB * CH
NP = 10240
ROWS_PER_SUB = NP // NSUB

IDXG = 126
PW1 = 80
PW2 = 128

_HIGH = jax.lax.Precision.HIGHEST


def _dot(a, b):
    return jax.lax.dot_general(a, b, (((1,), (0,)), ((), ())),
                               precision=_HIGH,
                               preferred_element_type=jnp.float32)


def _gelu(h):
    return 0.5 * h * (1.0 + lax.erf(h * 0.7071067811865476))


def _stage_a1(x, iso_idx_r, iso_embed):
    def body(x_ref, idx_ref, emb_ref, q0_ref, q1_ref, q2_ref, q3_ref):
        xb = x_ref[...]
        idx = jnp.reshape(idx_ref[0, 0, :], (ROWS, 1))
        onehot = (lax.broadcasted_iota(jnp.int32, (ROWS, EMB_N), 1)
                  == idx).astype(jnp.float32)
        e = _dot(onehot, emb_ref[...])
        q0_ref[...] = xb[:, 0:80]
        q1_ref[...] = xb[:, 80:160]
        q2_ref[...] = xb[:, 160:240]
        q3_ref[...] = jnp.concatenate([xb[:, 240:256], e], axis=1)

    return pl.pallas_call(
        body,
        grid=(GRID,),
        in_specs=[
            pl.BlockSpec((ROWS, D), lambda i: (i, 0)),
            pl.BlockSpec((1, 1, ROWS), lambda i: (i, 0, 0)),
            pl.BlockSpec((EMB_N, EMB_D), lambda i: (0, 0)),
        ],
        out_specs=[pl.BlockSpec((ROWS, PW1), lambda i: (i, 0))] * 4,
        out_shape=[jax.ShapeDtypeStruct((N, PW1), jnp.float32)] * 4,
    )(x, iso_idx_r, iso_embed)


def _stage_a2(q0, q1, q2, q3, W1r):
    def body(q0_ref, q1_ref, q2_ref, q3_ref, w_ref, o_ref):
        o_ref[...] = (_dot(q0_ref[...], w_ref[0:80, :])
                      + _dot(q1_ref[...], w_ref[80:160, :])
                      + _dot(q2_ref[...], w_ref[160:240, :])
                      + _dot(q3_ref[...], w_ref[240:320, :]))

    return pl.pallas_call(
        body,
        grid=(GRID,),
        in_specs=[pl.BlockSpec((ROWS, PW1), lambda i: (i, 0))] * 4
        + [pl.BlockSpec((TIN, HID), lambda i: (0, 0))],
        out_specs=pl.BlockSpec((ROWS, HID), lambda i: (i, 0)),
        out_shape=jax.ShapeDtypeStruct((N, HID), jnp.float32),
    )(q0, q1, q2, q3, W1r)


def _stage_b1(b0, b1, b2, b3, deg16, xw1r, W1l, b1l_r):
    def body(b0_ref, b1_ref, b2_ref, b3_ref, deg_ref, xw_ref, w_ref,
             bias_ref, p0_ref, p1_ref, p2_ref, p3_ref):
        inv = 1.0 / jnp.maximum(deg_ref[:, 0:1], 1.0)
        h = (_dot(b0_ref[...] * inv, w_ref[0:80, :])
             + _dot(b1_ref[...] * inv, w_ref[80:160, :])
             + _dot(b2_ref[...] * inv, w_ref[160:240, :])
             + _dot(b3_ref[...] * inv, w_ref[240:320, :])
             + xw_ref[...] + bias_ref[...])
        h = _gelu(h)
        p0_ref[...] = h[:, 0:128]
        p1_ref[...] = h[:, 128:256]
        p2_ref[...] = h[:, 256:384]
        p3_ref[...] = h[:, 384:512]

    return pl.pallas_call(
        body,
        grid=(GRID,),
        in_specs=[pl.BlockSpec((ROWS, PW1), lambda i: (i, 0))] * 4
        + [
            pl.BlockSpec((ROWS, 16), lambda i: (i, 0)),
            pl.BlockSpec((ROWS, HID), lambda i: (i, 0)),
            pl.BlockSpec((TIN, HID), lambda i: (0, 0)),
            pl.BlockSpec((1, HID), lambda i: (0, 0)),
        ],
        out_specs=[pl.BlockSpec((ROWS, PW2), lambda i: (i, 0))] * 4,
        out_shape=[jax.ShapeDtypeStruct((N, PW2), jnp.float32)] * 4,
    )(b0, b1, b2, b3, deg16, xw1r, W1l, b1l_r)


def _stage_b2(p0, p1, p2, p3, W2r):
    def body(p0_ref, p1_ref, p2_ref, p3_ref, w_ref, o_ref):
        o_ref[...] = (_dot(p0_ref[...], w_ref[0:128, :])
                      + _dot(p1_ref[...], w_ref[128:256, :])
                      + _dot(p2_ref[...], w_ref[256:384, :])
                      + _dot(p3_ref[...], w_ref[384:512, :]))

    return pl.pallas_call(
        body,
        grid=(GRID,),
        in_specs=[pl.BlockSpec((ROWS, PW2), lambda i: (i, 0))] * 4
        + [pl.BlockSpec((HID, HID), lambda i: (0, 0))],
        out_specs=pl.BlockSpec((ROWS, HID), lambda i: (i, 0)),
        out_shape=jax.ShapeDtypeStruct((N, HID), jnp.float32),
    )(p0, p1, p2, p3, W2r)


def _stage_c(a0, a1, a2, a3, deg16, h1w2r, W2l, b2l_r, Wh1, bh1_r,
             ln_g_r, ln_b_r, Wh2, bh2_r):
    def body(a0_ref, a1_ref, a2_ref, a3_ref, deg_ref, xw_ref, w2l_ref,
             b2l_ref, wh1_ref, bh1_ref, g_ref, b_ref, wh2_ref, bh2_ref,
             o_ref):
        inv = 1.0 / jnp.maximum(deg_ref[:, 0:1], 1.0)
        h = (_dot(a0_ref[...] * inv, w2l_ref[0:128, :])
             + _dot(a1_ref[...] * inv, w2l_ref[128:256, :])
             + _dot(a2_ref[...] * inv, w2l_ref[256:384, :])
             + _dot(a3_ref[...] * inv, w2l_ref[384:512, :])
             + xw_ref[...] + b2l_ref[...])
        h = _gelu(h)
        t = _dot(h, wh1_ref[...]) + bh1_ref[...]
        mu = jnp.mean(t, axis=-1, keepdims=True)
        var = jnp.mean((t - mu) ** 2, axis=-1, keepdims=True)
        t = (t - mu) * lax.rsqrt(var + 1e-5) * g_ref[...] + b_ref[...]
        t = _gelu(t)
        o_ref[...] = _dot(t, wh2_ref[...]) + bh2_ref[...]

    return pl.pallas_call(
        body,
        grid=(GRID,),
        in_specs=[pl.BlockSpec((ROWS, PW2), lambda i: (i, 0))] * 4
        + [
            pl.BlockSpec((ROWS, 16), lambda i: (i, 0)),
            pl.BlockSpec((ROWS, HID), lambda i: (i, 0)),
            pl.BlockSpec((HID, HID), lambda i: (0, 0)),
            pl.BlockSpec((1, HID), lambda i: (0, 0)),
            pl.BlockSpec((HID, 64), lambda i: (0, 0)),
            pl.BlockSpec((1, 64), lambda i: (0, 0)),
            pl.BlockSpec((1, 64), lambda i: (0, 0)),
            pl.BlockSpec((1, 64), lambda i: (0, 0)),
            pl.BlockSpec((64, 10), lambda i: (0, 0)),
            pl.BlockSpec((1, 10), lambda i: (0, 0)),
        ],
        out_specs=pl.BlockSpec((ROWS, 10), lambda i: (i, 0)),
        out_shape=jax.ShapeDtypeStruct((N, 10), jnp.float32),
    )(a0, a1, a2, a3, deg16, h1w2r, W2l, b2l_r, Wh1, bh1_r,
      ln_g_r, ln_b_r, Wh2, bh2_r)


def _mesh():
    return plsc.VectorSubcoreMesh(core_axis_name="c", subcore_axis_name="s")


_SC_PARAMS = pltpu.CompilerParams(use_tc_tiling_on_sc=False)


def _edge_loop(h_r, acc_s, sidx_v, didx_v, msg_a, msg_b, sem_a, sem_b,
               nch, dacc_s=None, ones_v=None):
    def fire(j, buf, sem):
        pltpu.async_copy(h_r.at[sidx_v.at[j]], buf, sem)

    def wait(j, buf, sem):
        pltpu.make_async_copy(h_r.at[sidx_v.at[j]], buf, sem).wait()

    def scat(j, buf):
        pltpu.sync_copy(buf, acc_s.at[didx_v.at[j]], add=True)
        if dacc_s is not None:
            pltpu.sync_copy(ones_v, dacc_s.at[didx_v.at[j]], add=True)

    fire(0, msg_a, sem_a)
    fire(1, msg_b, sem_b)

    @pl.loop(0, (nch - 2) // 2)
    def _(t):
        j = 2 * t
        wait(j, msg_a, sem_a)
        scat(j, msg_a)
        fire(j + 2, msg_a, sem_a)
        wait(j + 1, msg_b, sem_b)
        scat(j + 1, msg_b)
        fire(j + 3, msg_b, sem_b)

    wait(nch - 2, msg_a, sem_a)
    scat(nch - 2, msg_a)
    wait(nch - 1, msg_b, sem_b)
    scat(nch - 1, msg_b)


def _sc_agg1(src2, dst2, q0, q1, q2, q3, zacc, z16, ones16):
    @functools.partial(
        pl.kernel,
        out_type=[jax.ShapeDtypeStruct((NP, PW1), jnp.float32)] * 4
        + [jax.ShapeDtypeStruct((NP, 16), jnp.float32)],
        mesh=_mesh(),
        compiler_params=_SC_PARAMS,
        scratch_types=[
            pltpu.VMEM_SHARED((NP, PW1), jnp.float32),
            pltpu.VMEM_SHARED((NP, 16), jnp.float32),
            pltpu.VMEM((NCH_SUB, CH), jnp.int32),
            pltpu.VMEM((NCH_SUB, CH), jnp.int32),
            pltpu.VMEM((CH, PW1), jnp.float32),
            pltpu.VMEM((CH, PW1), jnp.float32),
            pltpu.VMEM((CH, 16), jnp.float32),
            pltpu.SemaphoreType.DMA,
            pltpu.SemaphoreType.DMA,
        ],
    )
    def k(src_r, dst_r, q0_r, q1_r, q2_r, q3_r, zacc_r, z16_r, ones_r,
          o0_r, o1_r, o2_r, o3_r, deg_r,
          acc_s, dacc_s, sidx_v, didx_v, msg_a, msg_b, ones_v,
          sem_a, sem_b):
        c = lax.axis_index("c")
        s = lax.axis_index("s")
        rbase = s * ROWS_PER_SUB
        cbase = s * NCH_SUB
        pltpu.sync_copy(src_r.at[pl.ds(cbase, NCH_SUB)], sidx_v)
        pltpu.sync_copy(dst_r.at[pl.ds(cbase, NCH_SUB)], didx_v)
        pltpu.sync_copy(ones_r, ones_v)

        def one_pass(h_r, o_r, with_deg):
            pltpu.sync_copy(zacc_r, acc_s.at[pl.ds(rbase, ROWS_PER_SUB)])
            if with_deg:
                pltpu.sync_copy(z16_r, dacc_s.at[pl.ds(rbase, ROWS_PER_SUB)])
            plsc.subcore_barrier()
            if with_deg:
                _edge_loop(h_r, acc_s, sidx_v, didx_v, msg_a, msg_b,
                           sem_a, sem_b, NCH_SUB, dacc_s, ones_v)
            else:
                _edge_loop(h_r, acc_s, sidx_v, didx_v, msg_a, msg_b,
                           sem_a, sem_b, NCH_SUB)

            plsc.subcore_barrier()
            pltpu.sync_copy(acc_s.at[pl.ds(rbase, ROWS_PER_SUB)],
                            o_r.at[pl.ds(rbase, ROWS_PER_SUB)])
            if with_deg:
                pltpu.sync_copy(dacc_s.at[pl.ds(rbase, ROWS_PER_SUB)],
                                deg_r.at[pl.ds(rbase, ROWS_PER_SUB)])
            plsc.subcore_barrier()

        @pl.when(c == 0)
        def _():
            one_pass(q0_r, o0_r, True)
            one_pass(q1_r, o1_r, False)

        @pl.when(c == 1)
        def _():
            one_pass(q2_r, o2_r, False)
            one_pass(q3_r, o3_r, False)

    return k(src2, dst2, q0, q1, q2, q3, zacc, z16, ones16)


def _sc_agg2(src2, dst2, p0, p1, p2, p3, z128):
    @functools.partial(
        pl.kernel,
        out_type=[jax.ShapeDtypeStruct((NP, PW2), jnp.float32)] * 4,
        mesh=_mesh(),
        compiler_params=_SC_PARAMS,
        scratch_types=[
            pltpu.VMEM_SHARED((NP, PW2), jnp.float32),
            pltpu.VMEM((IDXG, CH), jnp.int32),
            pltpu.VMEM((IDXG, CH), jnp.int32),
            pltpu.VMEM((CH, PW2), jnp.float32),
            pltpu.VMEM((CH, PW2), jnp.float32),
            pltpu.SemaphoreType.DMA,
            pltpu.SemaphoreType.DMA,
        ],
    )
    def k(src_r, dst_r, p0_r, p1_r, p2_r, p3_r, z128_r,
          o0_r, o1_r, o2_r, o3_r,
          acc_s, sidx_v, didx_v, msg_a, msg_b, sem_a, sem_b):
        c = lax.axis_index("c")
        s = lax.axis_index("s")
        rbase = s * ROWS_PER_SUB
        cbase = s * NCH_SUB

        def one_pass(h_r, o_r):
            pltpu.sync_copy(z128_r, acc_s.at[pl.ds(rbase, ROWS_PER_SUB)])
            plsc.subcore_barrier()
            for g in range(NCH_SUB // IDXG):
                pltpu.sync_copy(src_r.at[pl.ds(cbase + g * IDXG, IDXG)],
                                sidx_v)
                pltpu.sync_copy(dst_r.at[pl.ds(cbase + g * IDXG, IDXG)],
                                didx_v)
                _edge_loop(h_r, acc_s, sidx_v, didx_v, msg_a, msg_b,
                           sem_a, sem_b, IDXG)
            plsc.subcore_barrier()
            pltpu.sync_copy(acc_s.at[pl.ds(rbase, ROWS_PER_SUB)],
                            o_r.at[pl.ds(rbase, ROWS_PER_SUB)])
            plsc.subcore_barrier()

        @pl.when(c == 0)
        def _():
            one_pass(p0_r, o0_r)
            one_pass(p1_r, o1_r)

        @pl.when(c == 1)
        def _():
            one_pass(p2_r, o2_r)
            one_pass(p3_r, o3_r)

    return k(src2, dst2, p0, p1, p2, p3, z128)


def kernel(x, edge_index, iso_idx, iso_embed, W1l, b1l, W1r, W2l, b2l, W2r,
           Wh1, bh1, ln_g, ln_b, Wh2, bh2):
    src = edge_index[0].astype(jnp.int32)
    dst = edge_index[1].astype(jnp.int32)
    src2 = jnp.concatenate(
        [src, jnp.zeros((E_PAD - E,), jnp.int32)]).reshape(E_PAD // CH, CH)
    pad_dst = N + jnp.arange(E_PAD - E, dtype=jnp.int32) % (NP - N)
    dst2 = jnp.concatenate([dst, pad_dst]).reshape(E_PAD // CH, CH)
    iso_idx_r = iso_idx.astype(jnp.int32).reshape(GRID, 1, ROWS)

    z80 = jnp.zeros((ROWS_PER_SUB, PW1), jnp.float32)
    z128 = jnp.zeros((ROWS_PER_SUB, PW2), jnp.float32)
    z16 = jnp.zeros((ROWS_PER_SUB, 16), jnp.float32)
    ones16 = jnp.ones((CH, 16), jnp.float32)

    q0, q1, q2, q3 = _stage_a1(x, iso_idx_r, iso_embed)
    b0, b1, b2, b3, deg16 = _sc_agg1(src2, dst2, q0, q1, q2, q3,
                                     z80, z16, ones16)
    xw1r = _stage_a2(q0, q1, q2, q3, W1r)
    p0, p1, p2, p3 = _stage_b1(b0, b1, b2, b3, deg16, xw1r, W1l,
                               b1l.reshape(1, HID))
    a0, a1, a2, a3 = _sc_agg2(src2, dst2, p0, p1, p2, p3, z128)
    h1w2r = _stage_b2(p0, p1, p2, p3, W2r)
    out = _stage_c(a0, a1, a2, a3, deg16, h1w2r, W2l, b2l.reshape(1, HID),
                   Wh1, bh1.reshape(1, 64), ln_g.reshape(1, 64),
                   ln_b.reshape(1, 64), Wh2, bh2.reshape(1, 10))
    return out


# speedup vs baseline: 1.7770x; 1.2630x over previous
import functools

import jax
import jax.numpy as jnp
from jax import lax
from jax.experimental import pallas as pl
from jax.experimental.pallas import tpu as pltpu
from jax.experimental.pallas import tpu_sc as plsc

N = 10000
E = 160000
D = 256
EMB_N = 32
EMB_D = 64
HID = 512
TIN = D + EMB_D

ROWS = 400
GRID = N // ROWS

NSUB = 16
CH = 80
NCH_SUB = 125
E_PAD = NSUB * NCH_SUB * CH
NP = 10240
ROWS_PER_SUB = NP // NSUB

IDXG = 125
PW1 = 80
PW2 = 128

_HIGH = jax.lax.Precision.HIGHEST


def _dot(a, b):
    return jax.lax.dot_general(a, b, (((1,), (0,)), ((), ())),
                               precision=_HIGH,
                               preferred_element_type=jnp.float32)


def _gelu(h):
    return 0.5 * h * (1.0 + lax.erf(h * 0.7071067811865476))


def _stage_a1(x, iso_idx_r, iso_embed):
    def body(x_ref, idx_ref, emb_ref, q0_ref, q1_ref, q2_ref, q3_ref):
        xb = x_ref[...]
        idx = jnp.reshape(idx_ref[0, 0, :], (ROWS, 1))
        onehot = (lax.broadcasted_iota(jnp.int32, (ROWS, EMB_N), 1)
                  == idx).astype(jnp.float32)
        e = _dot(onehot, emb_ref[...])
        q0_ref[...] = xb[:, 0:80]
        q1_ref[...] = xb[:, 80:160]
        q2_ref[...] = xb[:, 160:240]
        q3_ref[...] = jnp.concatenate([xb[:, 240:256], e], axis=1)

    return pl.pallas_call(
        body,
        grid=(GRID,),
        in_specs=[
            pl.BlockSpec((ROWS, D), lambda i: (i, 0)),
            pl.BlockSpec((1, 1, ROWS), lambda i: (i, 0, 0)),
            pl.BlockSpec((EMB_N, EMB_D), lambda i: (0, 0)),
        ],
        out_specs=[pl.BlockSpec((ROWS, PW1), lambda i: (i, 0))] * 4,
        out_shape=[jax.ShapeDtypeStruct((N, PW1), jnp.float32)] * 4,
    )(x, iso_idx_r, iso_embed)


def _stage_a2(q0, q1, q2, q3, W1r):
    def body(q0_ref, q1_ref, q2_ref, q3_ref, w_ref, o_ref):
        o_ref[...] = (_dot(q0_ref[...], w_ref[0:80, :])
                      + _dot(q1_ref[...], w_ref[80:160, :])
                      + _dot(q2_ref[...], w_ref[160:240, :])
                      + _dot(q3_ref[...], w_ref[240:320, :]))

    return pl.pallas_call(
        body,
        grid=(GRID,),
        in_specs=[pl.BlockSpec((ROWS, PW1), lambda i: (i, 0))] * 4
        + [pl.BlockSpec((TIN, HID), lambda i: (0, 0))],
        out_specs=pl.BlockSpec((ROWS, HID), lambda i: (i, 0)),
        out_shape=jax.ShapeDtypeStruct((N, HID), jnp.float32),
    )(q0, q1, q2, q3, W1r)


def _stage_b1(b0, b1, b2, b3, deg16, xw1r, W1l, b1l_r):
    def body(b0_ref, b1_ref, b2_ref, b3_ref, deg_ref, xw_ref, w_ref,
             bias_ref, p0_ref, p1_ref, p2_ref, p3_ref):
        inv = 1.0 / jnp.maximum(deg_ref[:, 0:1], 1.0)
        h = (_dot(b0_ref[...] * inv, w_ref[0:80, :])
             + _dot(b1_ref[...] * inv, w_ref[80:160, :])
             + _dot(b2_ref[...] * inv, w_ref[160:240, :])
             + _dot(b3_ref[...] * inv, w_ref[240:320, :])
             + xw_ref[...] + bias_ref[...])
        h = _gelu(h)
        p0_ref[...] = h[:, 0:128]
        p1_ref[...] = h[:, 128:256]
        p2_ref[...] = h[:, 256:384]
        p3_ref[...] = h[:, 384:512]

    return pl.pallas_call(
        body,
        grid=(GRID,),
        in_specs=[pl.BlockSpec((ROWS, PW1), lambda i: (i, 0))] * 4
        + [
            pl.BlockSpec((ROWS, 16), lambda i: (i, 0)),
            pl.BlockSpec((ROWS, HID), lambda i: (i, 0)),
            pl.BlockSpec((TIN, HID), lambda i: (0, 0)),
            pl.BlockSpec((1, HID), lambda i: (0, 0)),
        ],
        out_specs=[pl.BlockSpec((ROWS, PW2), lambda i: (i, 0))] * 4,
        out_shape=[jax.ShapeDtypeStruct((N, PW2), jnp.float32)] * 4,
    )(b0, b1, b2, b3, deg16, xw1r, W1l, b1l_r)


def _stage_b2(p0, p1, p2, p3, W2r):
    def body(p0_ref, p1_ref, p2_ref, p3_ref, w_ref, o_ref):
        o_ref[...] = (_dot(p0_ref[...], w_ref[0:128, :])
                      + _dot(p1_ref[...], w_ref[128:256, :])
                      + _dot(p2_ref[...], w_ref[256:384, :])
                      + _dot(p3_ref[...], w_ref[384:512, :]))

    return pl.pallas_call(
        body,
        grid=(GRID,),
        in_specs=[pl.BlockSpec((ROWS, PW2), lambda i: (i, 0))] * 4
        + [pl.BlockSpec((HID, HID), lambda i: (0, 0))],
        out_specs=pl.BlockSpec((ROWS, HID), lambda i: (i, 0)),
        out_shape=jax.ShapeDtypeStruct((N, HID), jnp.float32),
    )(p0, p1, p2, p3, W2r)


def _stage_c(a0, a1, a2, a3, deg16, h1w2r, W2l, b2l_r, Wh1, bh1_r,
             ln_g_r, ln_b_r, Wh2, bh2_r):
    def body(a0_ref, a1_ref, a2_ref, a3_ref, deg_ref, xw_ref, w2l_ref,
             b2l_ref, wh1_ref, bh1_ref, g_ref, b_ref, wh2_ref, bh2_ref,
             o_ref):
        inv = 1.0 / jnp.maximum(deg_ref[:, 0:1], 1.0)
        h = (_dot(a0_ref[...] * inv, w2l_ref[0:128, :])
             + _dot(a1_ref[...] * inv, w2l_ref[128:256, :])
             + _dot(a2_ref[...] * inv, w2l_ref[256:384, :])
             + _dot(a3_ref[...] * inv, w2l_ref[384:512, :])
             + xw_ref[...] + b2l_ref[...])
        h = _gelu(h)
        t = _dot(h, wh1_ref[...]) + bh1_ref[...]
        mu = jnp.mean(t, axis=-1, keepdims=True)
        var = jnp.mean((t - mu) ** 2, axis=-1, keepdims=True)
        t = (t - mu) * lax.rsqrt(var + 1e-5) * g_ref[...] + b_ref[...]
        t = _gelu(t)
        o_ref[...] = _dot(t, wh2_ref[...]) + bh2_ref[...]

    return pl.pallas_call(
        body,
        grid=(GRID,),
        in_specs=[pl.BlockSpec((ROWS, PW2), lambda i: (i, 0))] * 4
        + [
            pl.BlockSpec((ROWS, 16), lambda i: (i, 0)),
            pl.BlockSpec((ROWS, HID), lambda i: (i, 0)),
            pl.BlockSpec((HID, HID), lambda i: (0, 0)),
            pl.BlockSpec((1, HID), lambda i: (0, 0)),
            pl.BlockSpec((HID, 64), lambda i: (0, 0)),
            pl.BlockSpec((1, 64), lambda i: (0, 0)),
            pl.BlockSpec((1, 64), lambda i: (0, 0)),
            pl.BlockSpec((1, 64), lambda i: (0, 0)),
            pl.BlockSpec((64, 10), lambda i: (0, 0)),
            pl.BlockSpec((1, 10), lambda i: (0, 0)),
        ],
        out_specs=pl.BlockSpec((ROWS, 10), lambda i: (i, 0)),
        out_shape=jax.ShapeDtypeStruct((N, 10), jnp.float32),
    )(a0, a1, a2, a3, deg16, h1w2r, W2l, b2l_r, Wh1, bh1_r,
      ln_g_r, ln_b_r, Wh2, bh2_r)


def _mesh():
    return plsc.VectorSubcoreMesh(core_axis_name="c", subcore_axis_name="s")


_SC_PARAMS = pltpu.CompilerParams(use_tc_tiling_on_sc=False)


def _edge_loop(h_r, acc_s, sidx_v, didx_v, msg_a, msg_b, sem_a, sem_b,
               nch, dacc_s=None, ones_v=None):
    def fire(j, buf, sem):
        pltpu.async_copy(h_r.at[sidx_v.at[j]], buf, sem)

    def wait(j, buf, sem):
        pltpu.make_async_copy(h_r.at[sidx_v.at[j]], buf, sem).wait()

    def scat(j, buf):
        pltpu.sync_copy(buf, acc_s.at[didx_v.at[j]], add=True)
        if dacc_s is not None:
            pltpu.sync_copy(ones_v, dacc_s.at[didx_v.at[j]], add=True)

    fire(0, msg_a, sem_a)
    fire(1, msg_b, sem_b)

    @pl.loop(0, (nch - 3) // 2 if nch % 2 else (nch - 2) // 2)
    def _(t):
        j = 2 * t
        wait(j, msg_a, sem_a)
        scat(j, msg_a)
        fire(j + 2, msg_a, sem_a)
        wait(j + 1, msg_b, sem_b)
        scat(j + 1, msg_b)
        fire(j + 3, msg_b, sem_b)

    if nch % 2:
        wait(nch - 3, msg_a, sem_a)
        scat(nch - 3, msg_a)
        fire(nch - 1, msg_a, sem_a)
        wait(nch - 2, msg_b, sem_b)
        scat(nch - 2, msg_b)
        wait(nch - 1, msg_a, sem_a)
        scat(nch - 1, msg_a)
    else:
        wait(nch - 2, msg_a, sem_a)
        scat(nch - 2, msg_a)
        wait(nch - 1, msg_b, sem_b)
        scat(nch - 1, msg_b)


def _sc_agg1(src2, dst2, q0, q1, q2, q3, zacc, z16, ones16):
    @functools.partial(
        pl.kernel,
        out_type=[jax.ShapeDtypeStruct((NP, PW1), jnp.float32)] * 4
        + [jax.ShapeDtypeStruct((NP, 16), jnp.float32)],
        mesh=_mesh(),
        compiler_params=_SC_PARAMS,
        scratch_types=[
            pltpu.VMEM_SHARED((NP, PW1), jnp.float32),
            pltpu.VMEM_SHARED((NP, 16), jnp.float32),
            pltpu.VMEM((NCH_SUB, CH), jnp.int32),
            pltpu.VMEM((NCH_SUB, CH), jnp.int32),
            pltpu.VMEM((CH, PW1), jnp.float32),
            pltpu.VMEM((CH, PW1), jnp.float32),
            pltpu.VMEM((CH, 16), jnp.float32),
            pltpu.SemaphoreType.DMA,
            pltpu.SemaphoreType.DMA,
        ],
    )
    def k(src_r, dst_r, q0_r, q1_r, q2_r, q3_r, zacc_r, z16_r, ones_r,
          o0_r, o1_r, o2_r, o3_r, deg_r,
          acc_s, dacc_s, sidx_v, didx_v, msg_a, msg_b, ones_v,
          sem_a, sem_b):
        c = lax.axis_index("c")
        s = lax.axis_index("s")
        rbase = s * ROWS_PER_SUB
        cbase = s * NCH_SUB
        pltpu.sync_copy(src_r.at[pl.ds(cbase, NCH_SUB)], sidx_v)
        pltpu.sync_copy(dst_r.at[pl.ds(cbase, NCH_SUB)], didx_v)
        pltpu.sync_copy(ones_r, ones_v)

        def one_pass(h_r, o_r, with_deg):
            pltpu.sync_copy(zacc_r, acc_s.at[pl.ds(rbase, ROWS_PER_SUB)])
            if with_deg:
                pltpu.sync_copy(z16_r, dacc_s.at[pl.ds(rbase, ROWS_PER_SUB)])
            plsc.subcore_barrier()
            if with_deg:
                _edge_loop(h_r, acc_s, sidx_v, didx_v, msg_a, msg_b,
                           sem_a, sem_b, NCH_SUB, dacc_s, ones_v)
            else:
                _edge_loop(h_r, acc_s, sidx_v, didx_v, msg_a, msg_b,
                           sem_a, sem_b, NCH_SUB)

            plsc.subcore_barrier()
            pltpu.sync_copy(acc_s.at[pl.ds(rbase, ROWS_PER_SUB)],
                            o_r.at[pl.ds(rbase, ROWS_PER_SUB)])
            if with_deg:
                pltpu.sync_copy(dacc_s.at[pl.ds(rbase, ROWS_PER_SUB)],
                                deg_r.at[pl.ds(rbase, ROWS_PER_SUB)])
            plsc.subcore_barrier()

        @pl.when(c == 0)
        def _():
            one_pass(q0_r, o0_r, True)
            one_pass(q1_r, o1_r, False)

        @pl.when(c == 1)
        def _():
            one_pass(q2_r, o2_r, False)
            one_pass(q3_r, o3_r, False)

    return k(src2, dst2, q0, q1, q2, q3, zacc, z16, ones16)


def _sc_agg2(src2, dst2, p0, p1, p2, p3, z128):
    @functools.partial(
        pl.kernel,
        out_type=[jax.ShapeDtypeStruct((NP, PW2), jnp.float32)] * 4,
        mesh=_mesh(),
        compiler_params=_SC_PARAMS,
        scratch_types=[
            pltpu.VMEM_SHARED((NP, PW2), jnp.float32),
            pltpu.VMEM((IDXG, CH), jnp.int32),
            pltpu.VMEM((IDXG, CH), jnp.int32),
            pltpu.VMEM((CH, PW2), jnp.float32),
            pltpu.VMEM((CH, PW2), jnp.float32),
            pltpu.SemaphoreType.DMA,
            pltpu.SemaphoreType.DMA,
        ],
    )
    def k(src_r, dst_r, p0_r, p1_r, p2_r, p3_r, z128_r,
          o0_r, o1_r, o2_r, o3_r,
          acc_s, sidx_v, didx_v, msg_a, msg_b, sem_a, sem_b):
        c = lax.axis_index("c")
        s = lax.axis_index("s")
        rbase = s * ROWS_PER_SUB
        cbase = s * NCH_SUB

        def one_pass(h_r, o_r):
            pltpu.sync_copy(z128_r, acc_s.at[pl.ds(rbase, ROWS_PER_SUB)])
            plsc.subcore_barrier()
            for g in range(NCH_SUB // IDXG):
                pltpu.sync_copy(src_r.at[pl.ds(cbase + g * IDXG, IDXG)],
                                sidx_v)
                pltpu.sync_copy(dst_r.at[pl.ds(cbase + g * IDXG, IDXG)],
                                didx_v)
                _edge_loop(h_r, acc_s, sidx_v, didx_v, msg_a, msg_b,
                           sem_a, sem_b, IDXG)
            plsc.subcore_barrier()
            pltpu.sync_copy(acc_s.at[pl.ds(rbase, ROWS_PER_SUB)],
                            o_r.at[pl.ds(rbase, ROWS_PER_SUB)])
            plsc.subcore_barrier()

        @pl.when(c == 0)
        def _():
            one_pass(p0_r, o0_r)
            one_pass(p1_r, o1_r)

        @pl.when(c == 1)
        def _():
            one_pass(p2_r, o2_r)
            one_pass(p3_r, o3_r)

    return k(src2, dst2, p0, p1, p2, p3, z128)


def kernel(x, edge_index, iso_idx, iso_embed, W1l, b1l, W1r, W2l, b2l, W2r,
           Wh1, bh1, ln_g, ln_b, Wh2, bh2):
    src2 = edge_index[0].astype(jnp.int32).reshape(E // CH, CH)
    dst2 = edge_index[1].astype(jnp.int32).reshape(E // CH, CH)
    iso_idx_r = iso_idx.astype(jnp.int32).reshape(GRID, 1, ROWS)

    z80 = jnp.zeros((ROWS_PER_SUB, PW1), jnp.float32)
    z128 = jnp.zeros((ROWS_PER_SUB, PW2), jnp.float32)
    z16 = jnp.zeros((ROWS_PER_SUB, 16), jnp.float32)
    ones16 = jnp.ones((CH, 16), jnp.float32)

    q0, q1, q2, q3 = _stage_a1(x, iso_idx_r, iso_embed)
    b0, b1, b2, b3, deg16 = _sc_agg1(src2, dst2, q0, q1, q2, q3,
                                     z80, z16, ones16)
    xw1r = _stage_a2(q0, q1, q2, q3, W1r)
    p0, p1, p2, p3 = _stage_b1(b0, b1, b2, b3, deg16, xw1r, W1l,
                               b1l.reshape(1, HID))
    a0, a1, a2, a3 = _sc_agg2(src2, dst2, p0, p1, p2, p3, z128)
    h1w2r = _stage_b2(p0, p1, p2, p3, W2r)
    out = _stage_c(a0, a1, a2, a3, deg16, h1w2r, W2l, b2l.reshape(1, HID),
                   Wh1, bh1.reshape(1, 64), ln_g.reshape(1, 64),
                   ln_b.reshape(1, 64), Wh2, bh2.reshape(1, 10))
    return out


# speedup vs baseline: 2.1237x; 1.1951x over previous
import functools

import jax
import jax.numpy as jnp
from jax import lax
from jax.experimental import pallas as pl
from jax.experimental.pallas import tpu as pltpu
from jax.experimental.pallas import tpu_sc as plsc

N = 10000
E = 160000
D = 256
EMB_N = 32
EMB_D = 64
HID = 512
TIN = D + EMB_D

ROWS = 400
GRID = N // ROWS

NSUB = 16
CH = 80
NCH_SUB = 125
E_PAD = NSUB * NCH_SUB * CH
NP = 10240
ROWS_PER_SUB = NP // NSUB

IDXG = 125
PW1 = 80
PW2 = 128

_HIGH = jax.lax.Precision.DEFAULT


def _dot(a, b):
    return jax.lax.dot_general(a, b, (((1,), (0,)), ((), ())),
                               precision=_HIGH,
                               preferred_element_type=jnp.float32)


def _gelu(h):
    return 0.5 * h * (1.0 + lax.erf(h * 0.7071067811865476))


def _stage_a1(x, iso_idx_r, iso_embed):
    def body(x_ref, idx_ref, emb_ref, q0_ref, q1_ref, q2_ref, q3_ref):
        xb = x_ref[...]
        idx = jnp.reshape(idx_ref[0, 0, :], (ROWS, 1))
        onehot = (lax.broadcasted_iota(jnp.int32, (ROWS, EMB_N), 1)
                  == idx).astype(jnp.float32)
        e = _dot(onehot, emb_ref[...])
        q0_ref[...] = xb[:, 0:80]
        q1_ref[...] = xb[:, 80:160]
        q2_ref[...] = xb[:, 160:240]
        q3_ref[...] = jnp.concatenate([xb[:, 240:256], e], axis=1)

    return pl.pallas_call(
        body,
        grid=(GRID,),
        in_specs=[
            pl.BlockSpec((ROWS, D), lambda i: (i, 0)),
            pl.BlockSpec((1, 1, ROWS), lambda i: (i, 0, 0)),
            pl.BlockSpec((EMB_N, EMB_D), lambda i: (0, 0)),
        ],
        out_specs=[pl.BlockSpec((ROWS, PW1), lambda i: (i, 0))] * 4,
        out_shape=[jax.ShapeDtypeStruct((N, PW1), jnp.float32)] * 4,
    )(x, iso_idx_r, iso_embed)


def _stage_a2(q0, q1, q2, q3, W1r):
    def body(q0_ref, q1_ref, q2_ref, q3_ref, w_ref, o_ref):
        o_ref[...] = (_dot(q0_ref[...], w_ref[0:80, :])
                      + _dot(q1_ref[...], w_ref[80:160, :])
                      + _dot(q2_ref[...], w_ref[160:240, :])
                      + _dot(q3_ref[...], w_ref[240:320, :]))

    return pl.pallas_call(
        body,
        grid=(GRID,),
        in_specs=[pl.BlockSpec((ROWS, PW1), lambda i: (i, 0))] * 4
        + [pl.BlockSpec((TIN, HID), lambda i: (0, 0))],
        out_specs=pl.BlockSpec((ROWS, HID), lambda i: (i, 0)),
        out_shape=jax.ShapeDtypeStruct((N, HID), jnp.float32),
    )(q0, q1, q2, q3, W1r)


def _stage_b1(b0, b1, b2, b3, deg16, xw1r, W1l, b1l_r):
    def body(b0_ref, b1_ref, b2_ref, b3_ref, deg_ref, xw_ref, w_ref,
             bias_ref, p0_ref, p1_ref, p2_ref, p3_ref):
        inv = 1.0 / jnp.maximum(deg_ref[:, 0:1], 1.0)
        h = (_dot(b0_ref[...] * inv, w_ref[0:80, :])
             + _dot(b1_ref[...] * inv, w_ref[80:160, :])
             + _dot(b2_ref[...] * inv, w_ref[160:240, :])
             + _dot(b3_ref[...] * inv, w_ref[240:320, :])
             + xw_ref[...] + bias_ref[...])
        h = _gelu(h)
        p0_ref[...] = h[:, 0:128]
        p1_ref[...] = h[:, 128:256]
        p2_ref[...] = h[:, 256:384]
        p3_ref[...] = h[:, 384:512]

    return pl.pallas_call(
        body,
        grid=(GRID,),
        in_specs=[pl.BlockSpec((ROWS, PW1), lambda i: (i, 0))] * 4
        + [
            pl.BlockSpec((ROWS, 16), lambda i: (i, 0)),
            pl.BlockSpec((ROWS, HID), lambda i: (i, 0)),
            pl.BlockSpec((TIN, HID), lambda i: (0, 0)),
            pl.BlockSpec((1, HID), lambda i: (0, 0)),
        ],
        out_specs=[pl.BlockSpec((ROWS, PW2), lambda i: (i, 0))] * 4,
        out_shape=[jax.ShapeDtypeStruct((N, PW2), jnp.float32)] * 4,
    )(b0, b1, b2, b3, deg16, xw1r, W1l, b1l_r)


def _stage_b2(p0, p1, p2, p3, W2r):
    def body(p0_ref, p1_ref, p2_ref, p3_ref, w_ref, o_ref):
        o_ref[...] = (_dot(p0_ref[...], w_ref[0:128, :])
                      + _dot(p1_ref[...], w_ref[128:256, :])
                      + _dot(p2_ref[...], w_ref[256:384, :])
                      + _dot(p3_ref[...], w_ref[384:512, :]))

    return pl.pallas_call(
        body,
        grid=(GRID,),
        in_specs=[pl.BlockSpec((ROWS, PW2), lambda i: (i, 0))] * 4
        + [pl.BlockSpec((HID, HID), lambda i: (0, 0))],
        out_specs=pl.BlockSpec((ROWS, HID), lambda i: (i, 0)),
        out_shape=jax.ShapeDtypeStruct((N, HID), jnp.float32),
    )(p0, p1, p2, p3, W2r)


def _stage_c(a0, a1, a2, a3, deg16, h1w2r, W2l, b2l_r, Wh1, bh1_r,
             ln_g_r, ln_b_r, Wh2, bh2_r):
    def body(a0_ref, a1_ref, a2_ref, a3_ref, deg_ref, xw_ref, w2l_ref,
             b2l_ref, wh1_ref, bh1_ref, g_ref, b_ref, wh2_ref, bh2_ref,
             o_ref):
        inv = 1.0 / jnp.maximum(deg_ref[:, 0:1], 1.0)
        h = (_dot(a0_ref[...] * inv, w2l_ref[0:128, :])
             + _dot(a1_ref[...] * inv, w2l_ref[128:256, :])
             + _dot(a2_ref[...] * inv, w2l_ref[256:384, :])
             + _dot(a3_ref[...] * inv, w2l_ref[384:512, :])
             + xw_ref[...] + b2l_ref[...])
        h = _gelu(h)
        t = _dot(h, wh1_ref[...]) + bh1_ref[...]
        mu = jnp.mean(t, axis=-1, keepdims=True)
        var = jnp.mean((t - mu) ** 2, axis=-1, keepdims=True)
        t = (t - mu) * lax.rsqrt(var + 1e-5) * g_ref[...] + b_ref[...]
        t = _gelu(t)
        o_ref[...] = _dot(t, wh2_ref[...]) + bh2_ref[...]

    return pl.pallas_call(
        body,
        grid=(GRID,),
        in_specs=[pl.BlockSpec((ROWS, PW2), lambda i: (i, 0))] * 4
        + [
            pl.BlockSpec((ROWS, 16), lambda i: (i, 0)),
            pl.BlockSpec((ROWS, HID), lambda i: (i, 0)),
            pl.BlockSpec((HID, HID), lambda i: (0, 0)),
            pl.BlockSpec((1, HID), lambda i: (0, 0)),
            pl.BlockSpec((HID, 64), lambda i: (0, 0)),
            pl.BlockSpec((1, 64), lambda i: (0, 0)),
            pl.BlockSpec((1, 64), lambda i: (0, 0)),
            pl.BlockSpec((1, 64), lambda i: (0, 0)),
            pl.BlockSpec((64, 10), lambda i: (0, 0)),
            pl.BlockSpec((1, 10), lambda i: (0, 0)),
        ],
        out_specs=pl.BlockSpec((ROWS, 10), lambda i: (i, 0)),
        out_shape=jax.ShapeDtypeStruct((N, 10), jnp.float32),
    )(a0, a1, a2, a3, deg16, h1w2r, W2l, b2l_r, Wh1, bh1_r,
      ln_g_r, ln_b_r, Wh2, bh2_r)


def _mesh():
    return plsc.VectorSubcoreMesh(core_axis_name="c", subcore_axis_name="s")


_SC_PARAMS = pltpu.CompilerParams(use_tc_tiling_on_sc=False)


def _edge_loop(h_r, acc_s, sidx_v, didx_v, msg_a, msg_b, sem_a, sem_b,
               nch, dacc_s=None, ones_v=None):
    def fire(j, buf, sem):
        pltpu.async_copy(h_r.at[sidx_v.at[j]], buf, sem)

    def wait(j, buf, sem):
        pltpu.make_async_copy(h_r.at[sidx_v.at[j]], buf, sem).wait()

    def scat(j, buf):
        pltpu.sync_copy(buf, acc_s.at[didx_v.at[j]], add=True)
        if dacc_s is not None:
            pltpu.sync_copy(ones_v, dacc_s.at[didx_v.at[j]], add=True)

    fire(0, msg_a, sem_a)
    fire(1, msg_b, sem_b)

    @pl.loop(0, (nch - 3) // 2 if nch % 2 else (nch - 2) // 2)
    def _(t):
        j = 2 * t
        wait(j, msg_a, sem_a)
        scat(j, msg_a)
        fire(j + 2, msg_a, sem_a)
        wait(j + 1, msg_b, sem_b)
        scat(j + 1, msg_b)
        fire(j + 3, msg_b, sem_b)

    if nch % 2:
        wait(nch - 3, msg_a, sem_a)
        scat(nch - 3, msg_a)
        fire(nch - 1, msg_a, sem_a)
        wait(nch - 2, msg_b, sem_b)
        scat(nch - 2, msg_b)
        wait(nch - 1, msg_a, sem_a)
        scat(nch - 1, msg_a)
    else:
        wait(nch - 2, msg_a, sem_a)
        scat(nch - 2, msg_a)
        wait(nch - 1, msg_b, sem_b)
        scat(nch - 1, msg_b)


def _sc_agg1(src2, dst2, q0, q1, q2, q3, zacc, z16, ones16):
    @functools.partial(
        pl.kernel,
        out_type=[jax.ShapeDtypeStruct((NP, PW1), jnp.float32)] * 4
        + [jax.ShapeDtypeStruct((NP, 16), jnp.float32)],
        mesh=_mesh(),
        compiler_params=_SC_PARAMS,
        scratch_types=[
            pltpu.VMEM_SHARED((NP, PW1), jnp.float32),
            pltpu.VMEM_SHARED((NP, 16), jnp.float32),
            pltpu.VMEM((NCH_SUB, CH), jnp.int32),
            pltpu.VMEM((NCH_SUB, CH), jnp.int32),
            pltpu.VMEM((CH, PW1), jnp.float32),
            pltpu.VMEM((CH, PW1), jnp.float32),
            pltpu.VMEM((CH, 16), jnp.float32),
            pltpu.SemaphoreType.DMA,
            pltpu.SemaphoreType.DMA,
        ],
    )
    def k(src_r, dst_r, q0_r, q1_r, q2_r, q3_r, zacc_r, z16_r, ones_r,
          o0_r, o1_r, o2_r, o3_r, deg_r,
          acc_s, dacc_s, sidx_v, didx_v, msg_a, msg_b, ones_v,
          sem_a, sem_b):
        c = lax.axis_index("c")
        s = lax.axis_index("s")
        rbase = s * ROWS_PER_SUB
        cbase = s * NCH_SUB
        pltpu.sync_copy(src_r.at[pl.ds(cbase, NCH_SUB)], sidx_v)
        pltpu.sync_copy(dst_r.at[pl.ds(cbase, NCH_SUB)], didx_v)
        pltpu.sync_copy(ones_r, ones_v)

        def one_pass(h_r, o_r, with_deg):
            pltpu.sync_copy(zacc_r, acc_s.at[pl.ds(rbase, ROWS_PER_SUB)])
            if with_deg:
                pltpu.sync_copy(z16_r, dacc_s.at[pl.ds(rbase, ROWS_PER_SUB)])
            plsc.subcore_barrier()
            if with_deg:
                _edge_loop(h_r, acc_s, sidx_v, didx_v, msg_a, msg_b,
                           sem_a, sem_b, NCH_SUB, dacc_s, ones_v)
            else:
                _edge_loop(h_r, acc_s, sidx_v, didx_v, msg_a, msg_b,
                           sem_a, sem_b, NCH_SUB)

            plsc.subcore_barrier()
            pltpu.sync_copy(acc_s.at[pl.ds(rbase, ROWS_PER_SUB)],
                            o_r.at[pl.ds(rbase, ROWS_PER_SUB)])
            if with_deg:
                pltpu.sync_copy(dacc_s.at[pl.ds(rbase, ROWS_PER_SUB)],
                                deg_r.at[pl.ds(rbase, ROWS_PER_SUB)])
            plsc.subcore_barrier()

        @pl.when(c == 0)
        def _():
            one_pass(q0_r, o0_r, True)
            one_pass(q1_r, o1_r, False)

        @pl.when(c == 1)
        def _():
            one_pass(q2_r, o2_r, False)
            one_pass(q3_r, o3_r, False)

    return k(src2, dst2, q0, q1, q2, q3, zacc, z16, ones16)


def _sc_agg2(src2, dst2, p0, p1, p2, p3, z128):
    @functools.partial(
        pl.kernel,
        out_type=[jax.ShapeDtypeStruct((NP, PW2), jnp.float32)] * 4,
        mesh=_mesh(),
        compiler_params=_SC_PARAMS,
        scratch_types=[
            pltpu.VMEM_SHARED((NP, PW2), jnp.float32),
            pltpu.VMEM((IDXG, CH), jnp.int32),
            pltpu.VMEM((IDXG, CH), jnp.int32),
            pltpu.VMEM((CH, PW2), jnp.float32),
            pltpu.VMEM((CH, PW2), jnp.float32),
            pltpu.SemaphoreType.DMA,
            pltpu.SemaphoreType.DMA,
        ],
    )
    def k(src_r, dst_r, p0_r, p1_r, p2_r, p3_r, z128_r,
          o0_r, o1_r, o2_r, o3_r,
          acc_s, sidx_v, didx_v, msg_a, msg_b, sem_a, sem_b):
        c = lax.axis_index("c")
        s = lax.axis_index("s")
        rbase = s * ROWS_PER_SUB
        cbase = s * NCH_SUB

        def one_pass(h_r, o_r):
            pltpu.sync_copy(z128_r, acc_s.at[pl.ds(rbase, ROWS_PER_SUB)])
            plsc.subcore_barrier()
            for g in range(NCH_SUB // IDXG):
                pltpu.sync_copy(src_r.at[pl.ds(cbase + g * IDXG, IDXG)],
                                sidx_v)
                pltpu.sync_copy(dst_r.at[pl.ds(cbase + g * IDXG, IDXG)],
                                didx_v)
                _edge_loop(h_r, acc_s, sidx_v, didx_v, msg_a, msg_b,
                           sem_a, sem_b, IDXG)
            plsc.subcore_barrier()
            pltpu.sync_copy(acc_s.at[pl.ds(rbase, ROWS_PER_SUB)],
                            o_r.at[pl.ds(rbase, ROWS_PER_SUB)])
            plsc.subcore_barrier()

        @pl.when(c == 0)
        def _():
            one_pass(p0_r, o0_r)
            one_pass(p1_r, o1_r)

        @pl.when(c == 1)
        def _():
            one_pass(p2_r, o2_r)
            one_pass(p3_r, o3_r)

    return k(src2, dst2, p0, p1, p2, p3, z128)


def kernel(x, edge_index, iso_idx, iso_embed, W1l, b1l, W1r, W2l, b2l, W2r,
           Wh1, bh1, ln_g, ln_b, Wh2, bh2):
    src2 = edge_index[0].astype(jnp.int32).reshape(E // CH, CH)
    dst2 = edge_index[1].astype(jnp.int32).reshape(E // CH, CH)
    iso_idx_r = iso_idx.astype(jnp.int32).reshape(GRID, 1, ROWS)

    z80 = jnp.zeros((ROWS_PER_SUB, PW1), jnp.float32)
    z128 = jnp.zeros((ROWS_PER_SUB, PW2), jnp.float32)
    z16 = jnp.zeros((ROWS_PER_SUB, 16), jnp.float32)
    ones16 = jnp.ones((CH, 16), jnp.float32)

    q0, q1, q2, q3 = _stage_a1(x, iso_idx_r, iso_embed)
    b0, b1, b2, b3, deg16 = _sc_agg1(src2, dst2, q0, q1, q2, q3,
                                     z80, z16, ones16)
    xw1r = _stage_a2(q0, q1, q2, q3, W1r)
    p0, p1, p2, p3 = _stage_b1(b0, b1, b2, b3, deg16, xw1r, W1l,
                               b1l.reshape(1, HID))
    a0, a1, a2, a3 = _sc_agg2(src2, dst2, p0, p1, p2, p3, z128)
    h1w2r = _stage_b2(p0, p1, p2, p3, W2r)
    out = _stage_c(a0, a1, a2, a3, deg16, h1w2r, W2l, b2l.reshape(1, HID),
                   Wh1, bh1.reshape(1, 64), ln_g.reshape(1, 64),
                   ln_b.reshape(1, 64), Wh2, bh2.reshape(1, 10))
    return out


# speedup vs baseline: 2.3149x; 1.0901x over previous
import functools

import jax
import jax.numpy as jnp
from jax import lax
from jax.experimental import pallas as pl
from jax.experimental.pallas import tpu as pltpu
from jax.experimental.pallas import tpu_sc as plsc

N = 10000
E = 160000
D = 256
EMB_N = 32
EMB_D = 64
HID = 512
TIN = D + EMB_D

ROWS = 400
GRID = N // ROWS

NSUB = 16
CH = 80
NCH_SUB = 125
E_PAD = NSUB * NCH_SUB * CH
NP = 10240
ROWS_PER_SUB = NP // NSUB

IDXG = 125
PW1 = 80
PW2 = 128

_HIGH = jax.lax.Precision.DEFAULT


def _dot(a, b):
    return jax.lax.dot_general(a, b, (((1,), (0,)), ((), ())),
                               precision=_HIGH,
                               preferred_element_type=jnp.float32)


def _gelu(h):
    return 0.5 * h * (1.0 + lax.erf(h * 0.7071067811865476))


def _stage_a1(x, iso_idx_r, iso_embed):
    def body(x_ref, idx_ref, emb_ref, q0_ref, q1_ref, q2_ref, q3_ref):
        xb = x_ref[...]
        idx = jnp.reshape(idx_ref[0, 0, :], (ROWS, 1))
        onehot = (lax.broadcasted_iota(jnp.int32, (ROWS, EMB_N), 1)
                  == idx).astype(jnp.float32)
        e = _dot(onehot, emb_ref[...])
        q0_ref[...] = xb[:, 0:80]
        q1_ref[...] = xb[:, 80:160]
        q2_ref[...] = xb[:, 160:240]
        q3_ref[...] = jnp.concatenate([xb[:, 240:256], e], axis=1)

    return pl.pallas_call(
        body,
        grid=(GRID,),
        in_specs=[
            pl.BlockSpec((ROWS, D), lambda i: (i, 0)),
            pl.BlockSpec((1, 1, ROWS), lambda i: (i, 0, 0)),
            pl.BlockSpec((EMB_N, EMB_D), lambda i: (0, 0)),
        ],
        out_specs=[pl.BlockSpec((ROWS, PW1), lambda i: (i, 0))] * 4,
        out_shape=[jax.ShapeDtypeStruct((N, PW1), jnp.float32)] * 4,
    )(x, iso_idx_r, iso_embed)


def _stage_a2(q0, q1, q2, q3, W1r):
    def body(q0_ref, q1_ref, q2_ref, q3_ref, w_ref, o_ref):
        o_ref[...] = (_dot(q0_ref[...], w_ref[0:80, :])
                      + _dot(q1_ref[...], w_ref[80:160, :])
                      + _dot(q2_ref[...], w_ref[160:240, :])
                      + _dot(q3_ref[...], w_ref[240:320, :]))

    return pl.pallas_call(
        body,
        grid=(GRID,),
        in_specs=[pl.BlockSpec((ROWS, PW1), lambda i: (i, 0))] * 4
        + [pl.BlockSpec((TIN, HID), lambda i: (0, 0))],
        out_specs=pl.BlockSpec((ROWS, HID), lambda i: (i, 0)),
        out_shape=jax.ShapeDtypeStruct((N, HID), jnp.float32),
    )(q0, q1, q2, q3, W1r)


def _stage_b1(b0, b1, b2, b3, deg16, xw1r, W1l, b1l_r):
    def body(b0_ref, b1_ref, b2_ref, b3_ref, deg_ref, xw_ref, w_ref,
             bias_ref, p0_ref, p1_ref, p2_ref, p3_ref):
        inv = 1.0 / jnp.maximum(deg_ref[:, 0:1], 1.0)
        h = (_dot(b0_ref[...] * inv, w_ref[0:80, :])
             + _dot(b1_ref[...] * inv, w_ref[80:160, :])
             + _dot(b2_ref[...] * inv, w_ref[160:240, :])
             + _dot(b3_ref[...] * inv, w_ref[240:320, :])
             + xw_ref[...] + bias_ref[...])
        h = _gelu(h)
        p0_ref[...] = h[:, 0:128]
        p1_ref[...] = h[:, 128:256]
        p2_ref[...] = h[:, 256:384]
        p3_ref[...] = h[:, 384:512]

    return pl.pallas_call(
        body,
        grid=(GRID,),
        in_specs=[pl.BlockSpec((ROWS, PW1), lambda i: (i, 0))] * 4
        + [
            pl.BlockSpec((ROWS, 16), lambda i: (i, 0)),
            pl.BlockSpec((ROWS, HID), lambda i: (i, 0)),
            pl.BlockSpec((TIN, HID), lambda i: (0, 0)),
            pl.BlockSpec((1, HID), lambda i: (0, 0)),
        ],
        out_specs=[pl.BlockSpec((ROWS, PW2), lambda i: (i, 0))] * 4,
        out_shape=[jax.ShapeDtypeStruct((N, PW2), jnp.float32)] * 4,
    )(b0, b1, b2, b3, deg16, xw1r, W1l, b1l_r)


def _stage_b2(p0, p1, p2, p3, W2r):
    def body(p0_ref, p1_ref, p2_ref, p3_ref, w_ref, o_ref):
        o_ref[...] = (_dot(p0_ref[...], w_ref[0:128, :])
                      + _dot(p1_ref[...], w_ref[128:256, :])
                      + _dot(p2_ref[...], w_ref[256:384, :])
                      + _dot(p3_ref[...], w_ref[384:512, :]))

    return pl.pallas_call(
        body,
        grid=(GRID,),
        in_specs=[pl.BlockSpec((ROWS, PW2), lambda i: (i, 0))] * 4
        + [pl.BlockSpec((HID, HID), lambda i: (0, 0))],
        out_specs=pl.BlockSpec((ROWS, HID), lambda i: (i, 0)),
        out_shape=jax.ShapeDtypeStruct((N, HID), jnp.float32),
    )(p0, p1, p2, p3, W2r)


def _stage_c(a0, a1, a2, a3, deg16, h1w2r, W2l, b2l_r, Wh1, bh1_r,
             ln_g_r, ln_b_r, Wh2, bh2_r):
    def body(a0_ref, a1_ref, a2_ref, a3_ref, deg_ref, xw_ref, w2l_ref,
             b2l_ref, wh1_ref, bh1_ref, g_ref, b_ref, wh2_ref, bh2_ref,
             o_ref):
        inv = 1.0 / jnp.maximum(deg_ref[:, 0:1], 1.0)
        h = (_dot(a0_ref[...] * inv, w2l_ref[0:128, :])
             + _dot(a1_ref[...] * inv, w2l_ref[128:256, :])
             + _dot(a2_ref[...] * inv, w2l_ref[256:384, :])
             + _dot(a3_ref[...] * inv, w2l_ref[384:512, :])
             + xw_ref[...] + b2l_ref[...])
        h = _gelu(h)
        t = _dot(h, wh1_ref[...]) + bh1_ref[...]
        mu = jnp.mean(t, axis=-1, keepdims=True)
        var = jnp.mean((t - mu) ** 2, axis=-1, keepdims=True)
        t = (t - mu) * lax.rsqrt(var + 1e-5) * g_ref[...] + b_ref[...]
        t = _gelu(t)
        o_ref[...] = _dot(t, wh2_ref[...]) + bh2_ref[...]

    return pl.pallas_call(
        body,
        grid=(GRID,),
        in_specs=[pl.BlockSpec((ROWS, PW2), lambda i: (i, 0))] * 4
        + [
            pl.BlockSpec((ROWS, 16), lambda i: (i, 0)),
            pl.BlockSpec((ROWS, HID), lambda i: (i, 0)),
            pl.BlockSpec((HID, HID), lambda i: (0, 0)),
            pl.BlockSpec((1, HID), lambda i: (0, 0)),
            pl.BlockSpec((HID, 64), lambda i: (0, 0)),
            pl.BlockSpec((1, 64), lambda i: (0, 0)),
            pl.BlockSpec((1, 64), lambda i: (0, 0)),
            pl.BlockSpec((1, 64), lambda i: (0, 0)),
            pl.BlockSpec((64, 10), lambda i: (0, 0)),
            pl.BlockSpec((1, 10), lambda i: (0, 0)),
        ],
        out_specs=pl.BlockSpec((ROWS, 10), lambda i: (i, 0)),
        out_shape=jax.ShapeDtypeStruct((N, 10), jnp.float32),
    )(a0, a1, a2, a3, deg16, h1w2r, W2l, b2l_r, Wh1, bh1_r,
      ln_g_r, ln_b_r, Wh2, bh2_r)


def _mesh():
    return plsc.VectorSubcoreMesh(core_axis_name="c", subcore_axis_name="s")


_SC_PARAMS = pltpu.CompilerParams(use_tc_tiling_on_sc=False)


def _edge_loop(h_r, acc_s, sidx_v, didx_v, bufs, sems, nch,
               dacc_s=None, ones_v=None):
    k = len(bufs)

    def fire(j, i):
        pltpu.async_copy(h_r.at[sidx_v.at[j]], bufs[i], sems[i])

    def wait(j, i):
        pltpu.make_async_copy(h_r.at[sidx_v.at[j]], bufs[i], sems[i]).wait()

    def scat(j, i):
        pltpu.sync_copy(bufs[i], acc_s.at[didx_v.at[j]], add=True)
        if dacc_s is not None:
            pltpu.sync_copy(ones_v, dacc_s.at[didx_v.at[j]], add=True)

    for i in range(k):
        fire(i, i)
    nmain = (nch - k) // k

    @pl.loop(0, nmain)
    def _(t):
        j = k * t
        for i in range(k):
            wait(j + i, i)
            scat(j + i, i)
            fire(j + i + k, i)

    for j in range(k * nmain, nch):
        wait(j, j % k)
        scat(j, j % k)
        if j + k < nch:
            fire(j + k, j % k)


def _sc_agg1(src2, dst2, q0, q1, q2, q3, zacc, z16, ones16):
    @functools.partial(
        pl.kernel,
        out_type=[jax.ShapeDtypeStruct((NP, PW1), jnp.float32)] * 4
        + [jax.ShapeDtypeStruct((NP, 16), jnp.float32)],
        mesh=_mesh(),
        compiler_params=_SC_PARAMS,
        scratch_types=[
            pltpu.VMEM_SHARED((NP, PW1), jnp.float32),
            pltpu.VMEM_SHARED((NP, 16), jnp.float32),
            pltpu.VMEM((NCH_SUB, CH), jnp.int32),
            pltpu.VMEM((NCH_SUB, CH), jnp.int32),
            pltpu.VMEM((CH, PW1), jnp.float32),
            pltpu.VMEM((CH, PW1), jnp.float32),
            pltpu.VMEM((CH, PW1), jnp.float32),
            pltpu.VMEM((CH, 16), jnp.float32),
            pltpu.SemaphoreType.DMA,
            pltpu.SemaphoreType.DMA,
            pltpu.SemaphoreType.DMA,
        ],
    )
    def k(src_r, dst_r, q0_r, q1_r, q2_r, q3_r, zacc_r, z16_r, ones_r,
          o0_r, o1_r, o2_r, o3_r, deg_r,
          acc_s, dacc_s, sidx_v, didx_v, msg_a, msg_b, msg_c, ones_v,
          sem_a, sem_b, sem_c):
        c = lax.axis_index("c")
        s = lax.axis_index("s")
        rbase = s * ROWS_PER_SUB
        cbase = s * NCH_SUB
        pltpu.sync_copy(src_r.at[pl.ds(cbase, NCH_SUB)], sidx_v)
        pltpu.sync_copy(dst_r.at[pl.ds(cbase, NCH_SUB)], didx_v)
        pltpu.sync_copy(ones_r, ones_v)

        def one_pass(h_r, o_r, with_deg):
            pltpu.sync_copy(zacc_r, acc_s.at[pl.ds(rbase, ROWS_PER_SUB)])
            if with_deg:
                pltpu.sync_copy(z16_r, dacc_s.at[pl.ds(rbase, ROWS_PER_SUB)])
            plsc.subcore_barrier()
            if with_deg:
                _edge_loop(h_r, acc_s, sidx_v, didx_v,
                           (msg_a, msg_b, msg_c), (sem_a, sem_b, sem_c),
                           NCH_SUB, dacc_s, ones_v)
            else:
                _edge_loop(h_r, acc_s, sidx_v, didx_v,
                           (msg_a, msg_b, msg_c), (sem_a, sem_b, sem_c),
                           NCH_SUB)

            plsc.subcore_barrier()
            pltpu.sync_copy(acc_s.at[pl.ds(rbase, ROWS_PER_SUB)],
                            o_r.at[pl.ds(rbase, ROWS_PER_SUB)])
            if with_deg:
                pltpu.sync_copy(dacc_s.at[pl.ds(rbase, ROWS_PER_SUB)],
                                deg_r.at[pl.ds(rbase, ROWS_PER_SUB)])

        @pl.when(c == 0)
        def _():
            one_pass(q0_r, o0_r, True)
            one_pass(q1_r, o1_r, False)

        @pl.when(c == 1)
        def _():
            one_pass(q2_r, o2_r, False)
            one_pass(q3_r, o3_r, False)

    return k(src2, dst2, q0, q1, q2, q3, zacc, z16, ones16)


def _sc_agg2(src2, dst2, p0, p1, p2, p3, z128):
    @functools.partial(
        pl.kernel,
        out_type=[jax.ShapeDtypeStruct((NP, PW2), jnp.float32)] * 4,
        mesh=_mesh(),
        compiler_params=_SC_PARAMS,
        scratch_types=[
            pltpu.VMEM_SHARED((NP, PW2), jnp.float32),
            pltpu.VMEM((IDXG, CH), jnp.int32),
            pltpu.VMEM((IDXG, CH), jnp.int32),
            pltpu.VMEM((CH, PW2), jnp.float32),
            pltpu.VMEM((CH, PW2), jnp.float32),
            pltpu.SemaphoreType.DMA,
            pltpu.SemaphoreType.DMA,
        ],
    )
    def k(src_r, dst_r, p0_r, p1_r, p2_r, p3_r, z128_r,
          o0_r, o1_r, o2_r, o3_r,
          acc_s, sidx_v, didx_v, msg_a, msg_b, sem_a, sem_b):
        c = lax.axis_index("c")
        s = lax.axis_index("s")
        rbase = s * ROWS_PER_SUB
        cbase = s * NCH_SUB

        def one_pass(h_r, o_r):
            pltpu.sync_copy(z128_r, acc_s.at[pl.ds(rbase, ROWS_PER_SUB)])
            plsc.subcore_barrier()
            for g in range(NCH_SUB // IDXG):
                pltpu.sync_copy(src_r.at[pl.ds(cbase + g * IDXG, IDXG)],
                                sidx_v)
                pltpu.sync_copy(dst_r.at[pl.ds(cbase + g * IDXG, IDXG)],
                                didx_v)
                _edge_loop(h_r, acc_s, sidx_v, didx_v,
                           (msg_a, msg_b), (sem_a, sem_b), IDXG)
            plsc.subcore_barrier()
            pltpu.sync_copy(acc_s.at[pl.ds(rbase, ROWS_PER_SUB)],
                            o_r.at[pl.ds(rbase, ROWS_PER_SUB)])

        @pl.when(c == 0)
        def _():
            one_pass(p0_r, o0_r)
            one_pass(p1_r, o1_r)

        @pl.when(c == 1)
        def _():
            one_pass(p2_r, o2_r)
            one_pass(p3_r, o3_r)

    return k(src2, dst2, p0, p1, p2, p3, z128)


def kernel(x, edge_index, iso_idx, iso_embed, W1l, b1l, W1r, W2l, b2l, W2r,
           Wh1, bh1, ln_g, ln_b, Wh2, bh2):
    src2 = edge_index[0].astype(jnp.int32).reshape(E // CH, CH)
    dst2 = edge_index[1].astype(jnp.int32).reshape(E // CH, CH)
    iso_idx_r = iso_idx.astype(jnp.int32).reshape(GRID, 1, ROWS)

    z80 = jnp.zeros((ROWS_PER_SUB, PW1), jnp.float32)
    z128 = jnp.zeros((ROWS_PER_SUB, PW2), jnp.float32)
    z16 = jnp.zeros((ROWS_PER_SUB, 16), jnp.float32)
    ones16 = jnp.ones((CH, 16), jnp.float32)

    q0, q1, q2, q3 = _stage_a1(x, iso_idx_r, iso_embed)
    b0, b1, b2, b3, deg16 = _sc_agg1(src2, dst2, q0, q1, q2, q3,
                                     z80, z16, ones16)
    xw1r = _stage_a2(q0, q1, q2, q3, W1r)
    p0, p1, p2, p3 = _stage_b1(b0, b1, b2, b3, deg16, xw1r, W1l,
                               b1l.reshape(1, HID))
    a0, a1, a2, a3 = _sc_agg2(src2, dst2, p0, p1, p2, p3, z128)
    h1w2r = _stage_b2(p0, p1, p2, p3, W2r)
    out = _stage_c(a0, a1, a2, a3, deg16, h1w2r, W2l, b2l.reshape(1, HID),
                   Wh1, bh1.reshape(1, 64), ln_g.reshape(1, 64),
                   ln_b.reshape(1, 64), Wh2, bh2.reshape(1, 10))
    return out


# speedup vs baseline: 2.4742x; 1.0688x over previous
import functools

import jax
import jax.numpy as jnp
from jax import lax
from jax.experimental import pallas as pl
from jax.experimental.pallas import tpu as pltpu
from jax.experimental.pallas import tpu_sc as plsc

N = 10000
E = 160000
D = 256
EMB_N = 32
EMB_D = 64
HID = 512
TIN = D + EMB_D

ROWS = 400
GRID = N // ROWS

NSUB = 16
CH = 80
NCH_SUB = 125
E_PAD = NSUB * NCH_SUB * CH
NP = 10240
ROWS_PER_SUB = NP // NSUB

IDXG = 63
PW1 = 80
PW2 = 128

_HIGH = jax.lax.Precision.DEFAULT


def _dot(a, b):
    return jax.lax.dot_general(a, b, (((1,), (0,)), ((), ())),
                               precision=_HIGH,
                               preferred_element_type=jnp.float32)


def _gelu(h):
    return 0.5 * h * (1.0 + lax.erf(h * 0.7071067811865476))


def _stage_a1(x, iso_idx_r, iso_embed):
    def body(x_ref, idx_ref, emb_ref, q0_ref, q1_ref, q2_ref, q3_ref):
        xb = x_ref[...]
        idx = jnp.reshape(idx_ref[0, 0, :], (ROWS, 1))
        onehot = (lax.broadcasted_iota(jnp.int32, (ROWS, EMB_N), 1)
                  == idx).astype(jnp.float32)
        e = _dot(onehot, emb_ref[...])
        q0_ref[...] = xb[:, 0:80]
        q1_ref[...] = xb[:, 80:160]
        q2_ref[...] = xb[:, 160:240]
        q3_ref[...] = jnp.concatenate([xb[:, 240:256], e], axis=1)

    return pl.pallas_call(
        body,
        grid=(GRID,),
        in_specs=[
            pl.BlockSpec((ROWS, D), lambda i: (i, 0)),
            pl.BlockSpec((1, 1, ROWS), lambda i: (i, 0, 0)),
            pl.BlockSpec((EMB_N, EMB_D), lambda i: (0, 0)),
        ],
        out_specs=[pl.BlockSpec((ROWS, PW1), lambda i: (i, 0))] * 4,
        out_shape=[jax.ShapeDtypeStruct((N, PW1), jnp.float32)] * 4,
    )(x, iso_idx_r, iso_embed)


def _stage_a2(q0, q1, q2, q3, W1r):
    def body(q0_ref, q1_ref, q2_ref, q3_ref, w_ref, o_ref):
        o_ref[...] = (_dot(q0_ref[...], w_ref[0:80, :])
                      + _dot(q1_ref[...], w_ref[80:160, :])
                      + _dot(q2_ref[...], w_ref[160:240, :])
                      + _dot(q3_ref[...], w_ref[240:320, :]))

    return pl.pallas_call(
        body,
        grid=(GRID,),
        in_specs=[pl.BlockSpec((ROWS, PW1), lambda i: (i, 0))] * 4
        + [pl.BlockSpec((TIN, HID), lambda i: (0, 0))],
        out_specs=pl.BlockSpec((ROWS, HID), lambda i: (i, 0)),
        out_shape=jax.ShapeDtypeStruct((N, HID), jnp.float32),
    )(q0, q1, q2, q3, W1r)


def _stage_b1(b0, b1, b2, b3, deg16, xw1r, W1l, b1l_r):
    def body(b0_ref, b1_ref, b2_ref, b3_ref, deg_ref, xw_ref, w_ref,
             bias_ref, p0_ref, p1_ref, p2_ref, p3_ref):
        inv = 1.0 / jnp.maximum(deg_ref[:, 0:1], 1.0)
        h = (_dot(b0_ref[...] * inv, w_ref[0:80, :])
             + _dot(b1_ref[...] * inv, w_ref[80:160, :])
             + _dot(b2_ref[...] * inv, w_ref[160:240, :])
             + _dot(b3_ref[...] * inv, w_ref[240:320, :])
             + xw_ref[...] + bias_ref[...])
        h = _gelu(h)
        p0_ref[...] = h[:, 0:128]
        p1_ref[...] = h[:, 128:256]
        p2_ref[...] = h[:, 256:384]
        p3_ref[...] = h[:, 384:512]

    return pl.pallas_call(
        body,
        grid=(GRID,),
        in_specs=[pl.BlockSpec((ROWS, PW1), lambda i: (i, 0))] * 4
        + [
            pl.BlockSpec((ROWS, 16), lambda i: (i, 0)),
            pl.BlockSpec((ROWS, HID), lambda i: (i, 0)),
            pl.BlockSpec((TIN, HID), lambda i: (0, 0)),
            pl.BlockSpec((1, HID), lambda i: (0, 0)),
        ],
        out_specs=[pl.BlockSpec((ROWS, PW2), lambda i: (i, 0))] * 4,
        out_shape=[jax.ShapeDtypeStruct((N, PW2), jnp.float32)] * 4,
    )(b0, b1, b2, b3, deg16, xw1r, W1l, b1l_r)


def _stage_b2(p0, p1, p2, p3, W2r):
    def body(p0_ref, p1_ref, p2_ref, p3_ref, w_ref, o_ref):
        o_ref[...] = (_dot(p0_ref[...], w_ref[0:128, :])
                      + _dot(p1_ref[...], w_ref[128:256, :])
                      + _dot(p2_ref[...], w_ref[256:384, :])
                      + _dot(p3_ref[...], w_ref[384:512, :]))

    return pl.pallas_call(
        body,
        grid=(GRID,),
        in_specs=[pl.BlockSpec((ROWS, PW2), lambda i: (i, 0))] * 4
        + [pl.BlockSpec((HID, HID), lambda i: (0, 0))],
        out_specs=pl.BlockSpec((ROWS, HID), lambda i: (i, 0)),
        out_shape=jax.ShapeDtypeStruct((N, HID), jnp.float32),
    )(p0, p1, p2, p3, W2r)


def _stage_c(a0, a1, a2, a3, deg16, h1w2r, W2l, b2l_r, Wh1, bh1_r,
             ln_g_r, ln_b_r, Wh2, bh2_r):
    def body(a0_ref, a1_ref, a2_ref, a3_ref, deg_ref, xw_ref, w2l_ref,
             b2l_ref, wh1_ref, bh1_ref, g_ref, b_ref, wh2_ref, bh2_ref,
             o_ref):
        inv = 1.0 / jnp.maximum(deg_ref[:, 0:1], 1.0)
        h = (_dot(a0_ref[...] * inv, w2l_ref[0:128, :])
             + _dot(a1_ref[...] * inv, w2l_ref[128:256, :])
             + _dot(a2_ref[...] * inv, w2l_ref[256:384, :])
             + _dot(a3_ref[...] * inv, w2l_ref[384:512, :])
             + xw_ref[...] + b2l_ref[...])
        h = _gelu(h)
        t = _dot(h, wh1_ref[...]) + bh1_ref[...]
        mu = jnp.mean(t, axis=-1, keepdims=True)
        var = jnp.mean((t - mu) ** 2, axis=-1, keepdims=True)
        t = (t - mu) * lax.rsqrt(var + 1e-5) * g_ref[...] + b_ref[...]
        t = _gelu(t)
        o_ref[...] = _dot(t, wh2_ref[...]) + bh2_ref[...]

    return pl.pallas_call(
        body,
        grid=(GRID,),
        in_specs=[pl.BlockSpec((ROWS, PW2), lambda i: (i, 0))] * 4
        + [
            pl.BlockSpec((ROWS, 16), lambda i: (i, 0)),
            pl.BlockSpec((ROWS, HID), lambda i: (i, 0)),
            pl.BlockSpec((HID, HID), lambda i: (0, 0)),
            pl.BlockSpec((1, HID), lambda i: (0, 0)),
            pl.BlockSpec((HID, 64), lambda i: (0, 0)),
            pl.BlockSpec((1, 64), lambda i: (0, 0)),
            pl.BlockSpec((1, 64), lambda i: (0, 0)),
            pl.BlockSpec((1, 64), lambda i: (0, 0)),
            pl.BlockSpec((64, 10), lambda i: (0, 0)),
            pl.BlockSpec((1, 10), lambda i: (0, 0)),
        ],
        out_specs=pl.BlockSpec((ROWS, 10), lambda i: (i, 0)),
        out_shape=jax.ShapeDtypeStruct((N, 10), jnp.float32),
    )(a0, a1, a2, a3, deg16, h1w2r, W2l, b2l_r, Wh1, bh1_r,
      ln_g_r, ln_b_r, Wh2, bh2_r)


def _mesh():
    return plsc.VectorSubcoreMesh(core_axis_name="c", subcore_axis_name="s")


_SC_PARAMS = pltpu.CompilerParams(use_tc_tiling_on_sc=False)


def _edge_loop(h_r, acc_s, sidx_v, didx_v, bufs, sems, nch,
               dacc_s=None, ones_v=None):
    k = len(bufs)

    def fire(j, i):
        pltpu.async_copy(h_r.at[sidx_v.at[j]], bufs[i], sems[i])

    def wait(j, i):
        pltpu.make_async_copy(h_r.at[sidx_v.at[j]], bufs[i], sems[i]).wait()

    def scat(j, i):
        pltpu.sync_copy(bufs[i], acc_s.at[didx_v.at[j]], add=True)
        if dacc_s is not None:
            pltpu.sync_copy(ones_v, dacc_s.at[didx_v.at[j]], add=True)

    for i in range(k):
        fire(i, i)
    nmain = (nch - k) // k

    @pl.loop(0, nmain)
    def _(t):
        j = k * t
        for i in range(k):
            wait(j + i, i)
            scat(j + i, i)
            fire(j + i + k, i)

    for j in range(k * nmain, nch):
        wait(j, j % k)
        scat(j, j % k)
        if j + k < nch:
            fire(j + k, j % k)


def _sc_agg1(src2, dst2, q0, q1, q2, q3, zacc, z16, ones16):
    @functools.partial(
        pl.kernel,
        out_type=[jax.ShapeDtypeStruct((NP, PW1), jnp.float32)] * 4
        + [jax.ShapeDtypeStruct((NP, 16), jnp.float32)],
        mesh=_mesh(),
        compiler_params=_SC_PARAMS,
        scratch_types=[
            pltpu.VMEM_SHARED((NP, PW1), jnp.float32),
            pltpu.VMEM_SHARED((NP, 16), jnp.float32),
            pltpu.VMEM((NCH_SUB, CH), jnp.int32),
            pltpu.VMEM((NCH_SUB, CH), jnp.int32),
            pltpu.VMEM((CH, PW1), jnp.float32),
            pltpu.VMEM((CH, PW1), jnp.float32),
            pltpu.VMEM((CH, PW1), jnp.float32),
            pltpu.VMEM((CH, 16), jnp.float32),
            pltpu.SemaphoreType.DMA,
            pltpu.SemaphoreType.DMA,
            pltpu.SemaphoreType.DMA,
        ],
    )
    def k(src_r, dst_r, q0_r, q1_r, q2_r, q3_r, zacc_r, z16_r, ones_r,
          o0_r, o1_r, o2_r, o3_r, deg_r,
          acc_s, dacc_s, sidx_v, didx_v, msg_a, msg_b, msg_c, ones_v,
          sem_a, sem_b, sem_c):
        c = lax.axis_index("c")
        s = lax.axis_index("s")
        rbase = s * ROWS_PER_SUB
        cbase = s * NCH_SUB
        pltpu.sync_copy(src_r.at[pl.ds(cbase, NCH_SUB)], sidx_v)
        pltpu.sync_copy(dst_r.at[pl.ds(cbase, NCH_SUB)], didx_v)
        pltpu.sync_copy(ones_r, ones_v)

        def one_pass(h_r, o_r, with_deg):
            pltpu.sync_copy(zacc_r, acc_s.at[pl.ds(rbase, ROWS_PER_SUB)])
            if with_deg:
                pltpu.sync_copy(z16_r, dacc_s.at[pl.ds(rbase, ROWS_PER_SUB)])
            plsc.subcore_barrier()
            if with_deg:
                _edge_loop(h_r, acc_s, sidx_v, didx_v,
                           (msg_a, msg_b, msg_c), (sem_a, sem_b, sem_c),
                           NCH_SUB, dacc_s, ones_v)
            else:
                _edge_loop(h_r, acc_s, sidx_v, didx_v,
                           (msg_a, msg_b, msg_c), (sem_a, sem_b, sem_c),
                           NCH_SUB)

            plsc.subcore_barrier()
            pltpu.sync_copy(acc_s.at[pl.ds(rbase, ROWS_PER_SUB)],
                            o_r.at[pl.ds(rbase, ROWS_PER_SUB)])
            if with_deg:
                pltpu.sync_copy(dacc_s.at[pl.ds(rbase, ROWS_PER_SUB)],
                                deg_r.at[pl.ds(rbase, ROWS_PER_SUB)])

        @pl.when(c == 0)
        def _():
            one_pass(q0_r, o0_r, True)
            one_pass(q1_r, o1_r, False)

        @pl.when(c == 1)
        def _():
            one_pass(q2_r, o2_r, False)
            one_pass(q3_r, o3_r, False)

    return k(src2, dst2, q0, q1, q2, q3, zacc, z16, ones16)


def _sc_agg2(src2, dst2, p0, p1, p2, p3, z128):
    @functools.partial(
        pl.kernel,
        out_type=[jax.ShapeDtypeStruct((NP, PW2), jnp.float32)] * 4,
        mesh=_mesh(),
        compiler_params=_SC_PARAMS,
        scratch_types=[
            pltpu.VMEM_SHARED((NP, PW2), jnp.float32),
            pltpu.VMEM((IDXG, CH), jnp.int32),
            pltpu.VMEM((IDXG, CH), jnp.int32),
            pltpu.VMEM((CH, PW2), jnp.float32),
            pltpu.VMEM((CH, PW2), jnp.float32),
            pltpu.VMEM((CH, PW2), jnp.float32),
            pltpu.SemaphoreType.DMA,
            pltpu.SemaphoreType.DMA,
            pltpu.SemaphoreType.DMA,
        ],
    )
    def k(src_r, dst_r, p0_r, p1_r, p2_r, p3_r, z128_r,
          o0_r, o1_r, o2_r, o3_r,
          acc_s, sidx_v, didx_v, msg_a, msg_b, msg_c,
          sem_a, sem_b, sem_c):
        c = lax.axis_index("c")
        s = lax.axis_index("s")
        rbase = s * ROWS_PER_SUB
        cbase = s * NCH_SUB

        def one_pass(h_r, o_r):
            pltpu.sync_copy(z128_r, acc_s.at[pl.ds(rbase, ROWS_PER_SUB)])
            plsc.subcore_barrier()
            for off, cnt in ((0, IDXG), (IDXG, NCH_SUB - IDXG)):
                pltpu.sync_copy(src_r.at[pl.ds(cbase + off, cnt)],
                                sidx_v.at[pl.ds(0, cnt)])
                pltpu.sync_copy(dst_r.at[pl.ds(cbase + off, cnt)],
                                didx_v.at[pl.ds(0, cnt)])
                _edge_loop(h_r, acc_s, sidx_v, didx_v,
                           (msg_a, msg_b, msg_c), (sem_a, sem_b, sem_c),
                           cnt)
            plsc.subcore_barrier()
            pltpu.sync_copy(acc_s.at[pl.ds(rbase, ROWS_PER_SUB)],
                            o_r.at[pl.ds(rbase, ROWS_PER_SUB)])

        @pl.when(c == 0)
        def _():
            one_pass(p0_r, o0_r)
            one_pass(p1_r, o1_r)

        @pl.when(c == 1)
        def _():
            one_pass(p2_r, o2_r)
            one_pass(p3_r, o3_r)

    return k(src2, dst2, p0, p1, p2, p3, z128)


def kernel(x, edge_index, iso_idx, iso_embed, W1l, b1l, W1r, W2l, b2l, W2r,
           Wh1, bh1, ln_g, ln_b, Wh2, bh2):
    src2 = edge_index[0].astype(jnp.int32).reshape(E // CH, CH)
    dst2 = edge_index[1].astype(jnp.int32).reshape(E // CH, CH)
    iso_idx_r = iso_idx.astype(jnp.int32).reshape(GRID, 1, ROWS)

    z80 = jnp.zeros((ROWS_PER_SUB, PW1), jnp.float32)
    z128 = jnp.zeros((ROWS_PER_SUB, PW2), jnp.float32)
    z16 = jnp.zeros((ROWS_PER_SUB, 16), jnp.float32)
    ones16 = jnp.ones((CH, 16), jnp.float32)

    q0, q1, q2, q3 = _stage_a1(x, iso_idx_r, iso_embed)
    b0, b1, b2, b3, deg16 = _sc_agg1(src2, dst2, q0, q1, q2, q3,
                                     z80, z16, ones16)
    xw1r = _stage_a2(q0, q1, q2, q3, W1r)
    p0, p1, p2, p3 = _stage_b1(b0, b1, b2, b3, deg16, xw1r, W1l,
                               b1l.reshape(1, HID))
    a0, a1, a2, a3 = _sc_agg2(src2, dst2, p0, p1, p2, p3, z128)
    h1w2r = _stage_b2(p0, p1, p2, p3, W2r)
    out = _stage_c(a0, a1, a2, a3, deg16, h1w2r, W2l, b2l.reshape(1, HID),
                   Wh1, bh1.reshape(1, 64), ln_g.reshape(1, 64),
                   ln_b.reshape(1, 64), Wh2, bh2.reshape(1, 10))
    return out


# speedup vs baseline: 2.4953x; 1.0086x over previous
import functools

import jax
import jax.numpy as jnp
from jax import lax
from jax.experimental import pallas as pl
from jax.experimental.pallas import tpu as pltpu
from jax.experimental.pallas import tpu_sc as plsc

N = 10000
E = 160000
D = 256
EMB_N = 32
EMB_D = 64
HID = 512
TIN = D + EMB_D

ROWS = 400
GRID = N // ROWS

NSUB = 16
CH = 80
NCH_SUB = 125
E_PAD = NSUB * NCH_SUB * CH
NP = 10240
ROWS_PER_SUB = NP // NSUB

IDXG = 32
PW1 = 80
PW2 = 128

_HIGH = jax.lax.Precision.DEFAULT


def _dot(a, b):
    return jax.lax.dot_general(a, b, (((1,), (0,)), ((), ())),
                               precision=_HIGH,
                               preferred_element_type=jnp.float32)


def _gelu(h):
    return 0.5 * h * (1.0 + lax.erf(h * 0.7071067811865476))


def _stage_a1(x, iso_idx_r, iso_embed):
    def body(x_ref, idx_ref, emb_ref, q0_ref, q1_ref, q2_ref, q3_ref):
        xb = x_ref[...]
        idx = jnp.reshape(idx_ref[0, 0, :], (ROWS, 1))
        onehot = (lax.broadcasted_iota(jnp.int32, (ROWS, EMB_N), 1)
                  == idx).astype(jnp.float32)
        e = _dot(onehot, emb_ref[...])
        q0_ref[...] = xb[:, 0:80]
        q1_ref[...] = xb[:, 80:160]
        q2_ref[...] = xb[:, 160:240]
        q3_ref[...] = jnp.concatenate([xb[:, 240:256], e], axis=1)

    return pl.pallas_call(
        body,
        grid=(GRID,),
        in_specs=[
            pl.BlockSpec((ROWS, D), lambda i: (i, 0)),
            pl.BlockSpec((1, 1, ROWS), lambda i: (i, 0, 0)),
            pl.BlockSpec((EMB_N, EMB_D), lambda i: (0, 0)),
        ],
        out_specs=[pl.BlockSpec((ROWS, PW1), lambda i: (i, 0))] * 4,
        out_shape=[jax.ShapeDtypeStruct((N, PW1), jnp.float32)] * 4,
    )(x, iso_idx_r, iso_embed)


def _stage_a2(q0, q1, q2, q3, W1r):
    def body(q0_ref, q1_ref, q2_ref, q3_ref, w_ref, o_ref):
        o_ref[...] = (_dot(q0_ref[...], w_ref[0:80, :])
                      + _dot(q1_ref[...], w_ref[80:160, :])
                      + _dot(q2_ref[...], w_ref[160:240, :])
                      + _dot(q3_ref[...], w_ref[240:320, :]))

    return pl.pallas_call(
        body,
        grid=(GRID,),
        in_specs=[pl.BlockSpec((ROWS, PW1), lambda i: (i, 0))] * 4
        + [pl.BlockSpec((TIN, HID), lambda i: (0, 0))],
        out_specs=pl.BlockSpec((ROWS, HID), lambda i: (i, 0)),
        out_shape=jax.ShapeDtypeStruct((N, HID), jnp.float32),
    )(q0, q1, q2, q3, W1r)


def _stage_b1(b0, b1, b2, b3, deg16, xw1r, W1l, b1l_r):
    def body(b0_ref, b1_ref, b2_ref, b3_ref, deg_ref, xw_ref, w_ref,
             bias_ref, p0_ref, p1_ref, p2_ref, p3_ref):
        inv = 1.0 / jnp.maximum(deg_ref[:, 0:1], 1.0)
        h = (_dot(b0_ref[...] * inv, w_ref[0:80, :])
             + _dot(b1_ref[...] * inv, w_ref[80:160, :])
             + _dot(b2_ref[...] * inv, w_ref[160:240, :])
             + _dot(b3_ref[...] * inv, w_ref[240:320, :])
             + xw_ref[...] + bias_ref[...])
        h = _gelu(h)
        p0_ref[...] = h[:, 0:128]
        p1_ref[...] = h[:, 128:256]
        p2_ref[...] = h[:, 256:384]
        p3_ref[...] = h[:, 384:512]

    return pl.pallas_call(
        body,
        grid=(GRID,),
        in_specs=[pl.BlockSpec((ROWS, PW1), lambda i: (i, 0))] * 4
        + [
            pl.BlockSpec((ROWS, 16), lambda i: (i, 0)),
            pl.BlockSpec((ROWS, HID), lambda i: (i, 0)),
            pl.BlockSpec((TIN, HID), lambda i: (0, 0)),
            pl.BlockSpec((1, HID), lambda i: (0, 0)),
        ],
        out_specs=[pl.BlockSpec((ROWS, PW2), lambda i: (i, 0))] * 4,
        out_shape=[jax.ShapeDtypeStruct((N, PW2), jnp.float32)] * 4,
    )(b0, b1, b2, b3, deg16, xw1r, W1l, b1l_r)


def _stage_b2(p0, p1, p2, p3, W2r):
    def body(p0_ref, p1_ref, p2_ref, p3_ref, w_ref, o_ref):
        o_ref[...] = (_dot(p0_ref[...], w_ref[0:128, :])
                      + _dot(p1_ref[...], w_ref[128:256, :])
                      + _dot(p2_ref[...], w_ref[256:384, :])
                      + _dot(p3_ref[...], w_ref[384:512, :]))

    return pl.pallas_call(
        body,
        grid=(GRID,),
        in_specs=[pl.BlockSpec((ROWS, PW2), lambda i: (i, 0))] * 4
        + [pl.BlockSpec((HID, HID), lambda i: (0, 0))],
        out_specs=pl.BlockSpec((ROWS, HID), lambda i: (i, 0)),
        out_shape=jax.ShapeDtypeStruct((N, HID), jnp.float32),
    )(p0, p1, p2, p3, W2r)


def _stage_c(a0, a1, a2, a3, deg16, h1w2r, W2l, b2l_r, Wh1, bh1_r,
             ln_g_r, ln_b_r, Wh2, bh2_r):
    def body(a0_ref, a1_ref, a2_ref, a3_ref, deg_ref, xw_ref, w2l_ref,
             b2l_ref, wh1_ref, bh1_ref, g_ref, b_ref, wh2_ref, bh2_ref,
             o_ref):
        inv = 1.0 / jnp.maximum(deg_ref[:, 0:1], 1.0)
        h = (_dot(a0_ref[...] * inv, w2l_ref[0:128, :])
             + _dot(a1_ref[...] * inv, w2l_ref[128:256, :])
             + _dot(a2_ref[...] * inv, w2l_ref[256:384, :])
             + _dot(a3_ref[...] * inv, w2l_ref[384:512, :])
             + xw_ref[...] + b2l_ref[...])
        h = _gelu(h)
        t = _dot(h, wh1_ref[...]) + bh1_ref[...]
        mu = jnp.mean(t, axis=-1, keepdims=True)
        var = jnp.mean((t - mu) ** 2, axis=-1, keepdims=True)
        t = (t - mu) * lax.rsqrt(var + 1e-5) * g_ref[...] + b_ref[...]
        t = _gelu(t)
        o_ref[...] = _dot(t, wh2_ref[...]) + bh2_ref[...]

    return pl.pallas_call(
        body,
        grid=(GRID,),
        in_specs=[pl.BlockSpec((ROWS, PW2), lambda i: (i, 0))] * 4
        + [
            pl.BlockSpec((ROWS, 16), lambda i: (i, 0)),
            pl.BlockSpec((ROWS, HID), lambda i: (i, 0)),
            pl.BlockSpec((HID, HID), lambda i: (0, 0)),
            pl.BlockSpec((1, HID), lambda i: (0, 0)),
            pl.BlockSpec((HID, 64), lambda i: (0, 0)),
            pl.BlockSpec((1, 64), lambda i: (0, 0)),
            pl.BlockSpec((1, 64), lambda i: (0, 0)),
            pl.BlockSpec((1, 64), lambda i: (0, 0)),
            pl.BlockSpec((64, 10), lambda i: (0, 0)),
            pl.BlockSpec((1, 10), lambda i: (0, 0)),
        ],
        out_specs=pl.BlockSpec((ROWS, 10), lambda i: (i, 0)),
        out_shape=jax.ShapeDtypeStruct((N, 10), jnp.float32),
    )(a0, a1, a2, a3, deg16, h1w2r, W2l, b2l_r, Wh1, bh1_r,
      ln_g_r, ln_b_r, Wh2, bh2_r)


def _mesh():
    return plsc.VectorSubcoreMesh(core_axis_name="c", subcore_axis_name="s")


_SC_PARAMS = pltpu.CompilerParams(use_tc_tiling_on_sc=False)


def _edge_loop(h_r, acc_s, sidx_v, didx_v, bufs, sems, nch,
               dacc_s=None, ones_v=None):
    k = len(bufs)

    def fire(j, i):
        pltpu.async_copy(h_r.at[sidx_v.at[j]], bufs[i], sems[i])

    def wait(j, i):
        pltpu.make_async_copy(h_r.at[sidx_v.at[j]], bufs[i], sems[i]).wait()

    def scat(j, i):
        pltpu.sync_copy(bufs[i], acc_s.at[didx_v.at[j]], add=True)
        if dacc_s is not None:
            pltpu.sync_copy(ones_v, dacc_s.at[didx_v.at[j]], add=True)

    for i in range(k):
        fire(i, i)
    nmain = (nch - k) // k

    @pl.loop(0, nmain)
    def _(t):
        j = k * t
        for i in range(k):
            wait(j + i, i)
            scat(j + i, i)
            fire(j + i + k, i)

    for j in range(k * nmain, nch):
        wait(j, j % k)
        scat(j, j % k)
        if j + k < nch:
            fire(j + k, j % k)


def _sc_agg1(src2, dst2, q0, q1, q2, q3, zacc, z16, ones16):
    @functools.partial(
        pl.kernel,
        out_type=[jax.ShapeDtypeStruct((NP, PW1), jnp.float32)] * 4
        + [jax.ShapeDtypeStruct((NP, 16), jnp.float32)],
        mesh=_mesh(),
        compiler_params=_SC_PARAMS,
        scratch_types=[
            pltpu.VMEM_SHARED((NP, PW1), jnp.float32),
            pltpu.VMEM_SHARED((NP, 16), jnp.float32),
            pltpu.VMEM((NCH_SUB, CH), jnp.int32),
            pltpu.VMEM((NCH_SUB, CH), jnp.int32),
            pltpu.VMEM((CH, PW1), jnp.float32),
            pltpu.VMEM((CH, PW1), jnp.float32),
            pltpu.VMEM((CH, PW1), jnp.float32),
            pltpu.VMEM((CH, PW1), jnp.float32),
            pltpu.VMEM((CH, 16), jnp.float32),
            pltpu.SemaphoreType.DMA,
            pltpu.SemaphoreType.DMA,
            pltpu.SemaphoreType.DMA,
            pltpu.SemaphoreType.DMA,
        ],
    )
    def k(src_r, dst_r, q0_r, q1_r, q2_r, q3_r, zacc_r, z16_r, ones_r,
          o0_r, o1_r, o2_r, o3_r, deg_r,
          acc_s, dacc_s, sidx_v, didx_v, msg_a, msg_b, msg_c, msg_d,
          ones_v, sem_a, sem_b, sem_c, sem_d):
        c = lax.axis_index("c")
        s = lax.axis_index("s")
        rbase = s * ROWS_PER_SUB
        cbase = s * NCH_SUB
        pltpu.sync_copy(src_r.at[pl.ds(cbase, NCH_SUB)], sidx_v)
        pltpu.sync_copy(dst_r.at[pl.ds(cbase, NCH_SUB)], didx_v)
        pltpu.sync_copy(ones_r, ones_v)

        def one_pass(h_r, o_r, with_deg):
            pltpu.sync_copy(zacc_r, acc_s.at[pl.ds(rbase, ROWS_PER_SUB)])
            if with_deg:
                pltpu.sync_copy(z16_r, dacc_s.at[pl.ds(rbase, ROWS_PER_SUB)])
            plsc.subcore_barrier()
            if with_deg:
                _edge_loop(h_r, acc_s, sidx_v, didx_v,
                           (msg_a, msg_b, msg_c, msg_d),
                           (sem_a, sem_b, sem_c, sem_d),
                           NCH_SUB, dacc_s, ones_v)
            else:
                _edge_loop(h_r, acc_s, sidx_v, didx_v,
                           (msg_a, msg_b, msg_c, msg_d),
                           (sem_a, sem_b, sem_c, sem_d),
                           NCH_SUB)

            plsc.subcore_barrier()
            pltpu.sync_copy(acc_s.at[pl.ds(rbase, ROWS_PER_SUB)],
                            o_r.at[pl.ds(rbase, ROWS_PER_SUB)])
            if with_deg:
                pltpu.sync_copy(dacc_s.at[pl.ds(rbase, ROWS_PER_SUB)],
                                deg_r.at[pl.ds(rbase, ROWS_PER_SUB)])

        @pl.when(c == 0)
        def _():
            one_pass(q0_r, o0_r, True)
            one_pass(q1_r, o1_r, False)

        @pl.when(c == 1)
        def _():
            one_pass(q2_r, o2_r, False)
            one_pass(q3_r, o3_r, False)

    return k(src2, dst2, q0, q1, q2, q3, zacc, z16, ones16)


def _sc_agg2(src2, dst2, p0, p1, p2, p3, z128):
    @functools.partial(
        pl.kernel,
        out_type=[jax.ShapeDtypeStruct((NP, PW2), jnp.float32)] * 4,
        mesh=_mesh(),
        compiler_params=_SC_PARAMS,
        scratch_types=[
            pltpu.VMEM_SHARED((NP, PW2), jnp.float32),
            pltpu.VMEM((IDXG, CH), jnp.int32),
            pltpu.VMEM((IDXG, CH), jnp.int32),
            pltpu.VMEM((CH, PW2), jnp.float32),
            pltpu.VMEM((CH, PW2), jnp.float32),
            pltpu.VMEM((CH, PW2), jnp.float32),
            pltpu.VMEM((CH, PW2), jnp.float32),
            pltpu.SemaphoreType.DMA,
            pltpu.SemaphoreType.DMA,
            pltpu.SemaphoreType.DMA,
            pltpu.SemaphoreType.DMA,
        ],
    )
    def k(src_r, dst_r, p0_r, p1_r, p2_r, p3_r, z128_r,
          o0_r, o1_r, o2_r, o3_r,
          acc_s, sidx_v, didx_v, msg_a, msg_b, msg_c, msg_d,
          sem_a, sem_b, sem_c, sem_d):
        c = lax.axis_index("c")
        s = lax.axis_index("s")
        rbase = s * ROWS_PER_SUB
        cbase = s * NCH_SUB

        def one_pass(h_r, o_r):
            pltpu.sync_copy(z128_r, acc_s.at[pl.ds(rbase, ROWS_PER_SUB)])
            plsc.subcore_barrier()
            groups = [(g * IDXG, min(IDXG, NCH_SUB - g * IDXG))
                      for g in range((NCH_SUB + IDXG - 1) // IDXG)]
            for off, cnt in groups:
                pltpu.sync_copy(src_r.at[pl.ds(cbase + off, cnt)],
                                sidx_v.at[pl.ds(0, cnt)])
                pltpu.sync_copy(dst_r.at[pl.ds(cbase + off, cnt)],
                                didx_v.at[pl.ds(0, cnt)])
                _edge_loop(h_r, acc_s, sidx_v, didx_v,
                           (msg_a, msg_b, msg_c, msg_d),
                           (sem_a, sem_b, sem_c, sem_d), cnt)
            plsc.subcore_barrier()
            pltpu.sync_copy(acc_s.at[pl.ds(rbase, ROWS_PER_SUB)],
                            o_r.at[pl.ds(rbase, ROWS_PER_SUB)])

        @pl.when(c == 0)
        def _():
            one_pass(p0_r, o0_r)
            one_pass(p1_r, o1_r)

        @pl.when(c == 1)
        def _():
            one_pass(p2_r, o2_r)
            one_pass(p3_r, o3_r)

    return k(src2, dst2, p0, p1, p2, p3, z128)


def kernel(x, edge_index, iso_idx, iso_embed, W1l, b1l, W1r, W2l, b2l, W2r,
           Wh1, bh1, ln_g, ln_b, Wh2, bh2):
    src2 = edge_index[0].astype(jnp.int32).reshape(E // CH, CH)
    dst2 = edge_index[1].astype(jnp.int32).reshape(E // CH, CH)
    iso_idx_r = iso_idx.astype(jnp.int32).reshape(GRID, 1, ROWS)

    z80 = jnp.zeros((ROWS_PER_SUB, PW1), jnp.float32)
    z128 = jnp.zeros((ROWS_PER_SUB, PW2), jnp.float32)
    z16 = jnp.zeros((ROWS_PER_SUB, 16), jnp.float32)
    ones16 = jnp.ones((CH, 16), jnp.float32)

    q0, q1, q2, q3 = _stage_a1(x, iso_idx_r, iso_embed)
    b0, b1, b2, b3, deg16 = _sc_agg1(src2, dst2, q0, q1, q2, q3,
                                     z80, z16, ones16)
    xw1r = _stage_a2(q0, q1, q2, q3, W1r)
    p0, p1, p2, p3 = _stage_b1(b0, b1, b2, b3, deg16, xw1r, W1l,
                               b1l.reshape(1, HID))
    a0, a1, a2, a3 = _sc_agg2(src2, dst2, p0, p1, p2, p3, z128)
    h1w2r = _stage_b2(p0, p1, p2, p3, W2r)
    out = _stage_c(a0, a1, a2, a3, deg16, h1w2r, W2l, b2l.reshape(1, HID),
                   Wh1, bh1.reshape(1, 64), ln_g.reshape(1, 64),
                   ln_b.reshape(1, 64), Wh2, bh2.reshape(1, 10))
    return out
